# Initial kernel scaffold; baseline (speedup 1.0000x reference)
import functools

import jax
import jax.numpy as jnp
from jax import lax
from jax.experimental import pallas as pl
from jax.experimental.pallas import tpu as pltpu
from jax.experimental.pallas import tpu_sc as plsc

N = 10000
D = 128
DD = 16
E = 320000
R_CUTOFF = 5.0
SPEED = 0.1

NC = 2
NS = 16
NW = NC * NS
C = 128
NCHUNK = E // C
CHUNKS_PER_W = -(-NCHUNK // NW)
ROWS_PER_TILE = N // NS

_MESH = plsc.VectorSubcoreMesh(core_axis_name="c", subcore_axis_name="s")


def _pq_body(h_ref, w1s_ref, w1d_ref, p_ref, q_ref):
    hb = h_ref[...]
    p_ref[...] = jnp.dot(hb, w1s_ref[...], preferred_element_type=jnp.float32)
    q_ref[...] = jnp.dot(hb, w1d_ref[...], preferred_element_type=jnp.float32)


def _pq(h, w1s, w1d):
    return pl.pallas_call(
        _pq_body,
        out_shape=[jax.ShapeDtypeStruct((N, D), jnp.float32),
                   jax.ShapeDtypeStruct((N, D), jnp.float32)],
    )(h, w1s, w1d)


def _gather_body(p_hbm, q_hbm, xflat_hbm, edges_hbm,
                 ps_hbm, qd_hbm, diff_hbm, d2_hbm,
                 xtile, srcv, dstv, psbuf, qdbuf, dbx, dby, dbz, d2b,
                 sem1, sem2):
    wid = lax.axis_index("s") * NC + lax.axis_index("c")
    pltpu.sync_copy(xflat_hbm, xtile)

    def chunk(j, carry):
        cid = wid + j * NW

        @pl.when(cid < NCHUNK)
        def _():
            off = cid * C
            pltpu.sync_copy(edges_hbm.at[0, pl.ds(off, C)], srcv)
            pltpu.sync_copy(edges_hbm.at[1, pl.ds(off, C)], dstv)
            cp1 = pltpu.async_copy(p_hbm.at[srcv], psbuf, sem1)
            cp2 = pltpu.async_copy(q_hbm.at[dstv], qdbuf, sem2)
            for k in range(C // 16):
                sl = pl.ds(k * 16, 16)
                s16 = srcv[sl] * 3
                d16 = dstv[sl] * 3
                dx = (plsc.load_gather(xtile, [d16])
                      - plsc.load_gather(xtile, [s16]))
                dy = (plsc.load_gather(xtile, [d16 + 1])
                      - plsc.load_gather(xtile, [s16 + 1]))
                dz = (plsc.load_gather(xtile, [d16 + 2])
                      - plsc.load_gather(xtile, [s16 + 2]))
                dbx[sl] = dx
                dby[sl] = dy
                dbz[sl] = dz
                d2b[sl] = dx * dx + dy * dy + dz * dz
            cp1.wait()
            cp2.wait()
            pltpu.sync_copy(psbuf, ps_hbm.at[pl.ds(off, C)])
            pltpu.sync_copy(qdbuf, qd_hbm.at[pl.ds(off, C)])
            pltpu.sync_copy(dbx, diff_hbm.at[0, pl.ds(off, C)])
            pltpu.sync_copy(dby, diff_hbm.at[1, pl.ds(off, C)])
            pltpu.sync_copy(dbz, diff_hbm.at[2, pl.ds(off, C)])
            pltpu.sync_copy(d2b, d2_hbm.at[pl.ds(off, C)])

        return carry

    lax.fori_loop(0, CHUNKS_PER_W, chunk, None)


def _gather(p, q, xflat, edges):
    f = functools.partial(
        pl.kernel,
        out_type=[jax.ShapeDtypeStruct((E, D), jnp.float32),
                  jax.ShapeDtypeStruct((E, D), jnp.float32),
                  jax.ShapeDtypeStruct((3, E), jnp.float32),
                  jax.ShapeDtypeStruct((E,), jnp.float32)],
        mesh=_MESH,
        scratch_types=[
            pltpu.VMEM((N * 3,), jnp.float32),
            pltpu.VMEM((C,), jnp.int32),
            pltpu.VMEM((C,), jnp.int32),
            pltpu.VMEM((C, D), jnp.float32),
            pltpu.VMEM((C, D), jnp.float32),
            pltpu.VMEM((C,), jnp.float32),
            pltpu.VMEM((C,), jnp.float32),
            pltpu.VMEM((C,), jnp.float32),
            pltpu.VMEM((C,), jnp.float32),
            pltpu.SemaphoreType.DMA,
            pltpu.SemaphoreType.DMA,
        ],
    )(_gather_body)
    return f(p, q, xflat, edges)


BE = 4000


def _edge_body(ps_ref, qd_ref, d2_ref, means_ref, inv2s2_ref,
               w1r_ref, b1_ref, w2_ref, b2_ref, wa_ref, ba_ref,
               wx1_ref, bx1_ref, wx2_ref,
               m_ref, coef_ref):
    d2 = d2_ref[...]
    dist = jnp.sqrt(d2 + 1e-12)
    valid = (dist < R_CUTOFF).astype(jnp.float32)
    delta = dist - means_ref[...]
    rbf = jnp.exp(-delta * delta * inv2s2_ref[...])
    u = (ps_ref[...] + qd_ref[...] + b1_ref[...]
         + jnp.dot(rbf, w1r_ref[...], preferred_element_type=jnp.float32))
    m1 = u * jax.nn.sigmoid(u)
    v = jnp.dot(m1, w2_ref[...], preferred_element_type=jnp.float32) + b2_ref[...]
    m2 = v * jax.nn.sigmoid(v)
    att = jax.nn.sigmoid(
        jnp.dot(m2, wa_ref[...], preferred_element_type=jnp.float32) + ba_ref[...])
    m_att = m2 * att
    m_ref[...] = m_att * valid
    g = jnp.dot(m_att, wx1_ref[...], preferred_element_type=jnp.float32) + bx1_ref[...]
    g = g * jax.nn.sigmoid(g)
    mag = jnp.tanh(jnp.dot(g, wx2_ref[...], preferred_element_type=jnp.float32))
    coef_ref[...] = SPEED * valid * mag / dist


def _edge(ps, qd, d2, means, inv2s2, w1r, b1, w2, b2, wa, ba, wx1, bx1, wx2):
    grid = E // BE
    full = lambda shape: pl.BlockSpec(shape, lambda i: (0, 0))
    return pl.pallas_call(
        _edge_body,
        grid=(grid,),
        in_specs=[
            pl.BlockSpec((BE, D), lambda i: (i, 0)),
            pl.BlockSpec((BE, D), lambda i: (i, 0)),
            pl.BlockSpec((BE, 1), lambda i: (i, 0)),
            full((1, DD)), full((1, DD)),
            full((DD, D)), full((1, D)), full((D, D)), full((1, D)),
            full((D, 1)), full((1, 1)),
            full((D, D)), full((1, D)), full((D, 1)),
        ],
        out_specs=[
            pl.BlockSpec((BE, D), lambda i: (i, 0)),
            pl.BlockSpec((BE, 1), lambda i: (i, 0)),
        ],
        out_shape=[jax.ShapeDtypeStruct((E, D), jnp.float32),
                   jax.ShapeDtypeStruct((E, 1), jnp.float32)],
    )(ps, qd, d2, means, inv2s2, w1r, b1, w2, b2, wa, ba, wx1, bx1, wx2)


XW = 16


def _scatter_body(m_hbm, coef_hbm, diff_hbm, edges_hbm, zm_hbm, zx_hbm,
                  msgp_hbm, xaccp_hbm,
                  msg_acc, xacc_sh,
                  dstv, mv, cv, dfx, dfy, dfz, dispbuf, zvm, zvx):
    cidx = lax.axis_index("c")
    sid = lax.axis_index("s")
    wid = sid * NC + cidx
    r0 = sid * ROWS_PER_TILE

    pltpu.sync_copy(zm_hbm, zvm)
    pltpu.sync_copy(zx_hbm, zvx)
    for i in range(5):
        pltpu.sync_copy(zvm, msg_acc.at[pl.ds(r0 + i * 125, 125)])
    pltpu.sync_copy(zvx, xacc_sh.at[pl.ds(r0, ROWS_PER_TILE)])
    pltpu.sync_copy(zx_hbm.at[pl.ds(0, C)], dispbuf)
    plsc.subcore_barrier()

    iota16 = lax.iota(jnp.int32, 16)

    def chunk(j, carry):
        cid = wid + j * NW

        @pl.when(cid < NCHUNK)
        def _():
            off = cid * C
            pltpu.sync_copy(edges_hbm.at[1, pl.ds(off, C)], dstv)
            pltpu.sync_copy(m_hbm.at[pl.ds(off, C)], mv)
            pltpu.sync_copy(coef_hbm.at[pl.ds(off, C)], cv)
            pltpu.sync_copy(diff_hbm.at[0, pl.ds(off, C)], dfx)
            pltpu.sync_copy(diff_hbm.at[1, pl.ds(off, C)], dfy)
            pltpu.sync_copy(diff_hbm.at[2, pl.ds(off, C)], dfz)
            for k in range(C // 16):
                sl = pl.ds(k * 16, 16)
                c16 = cv[sl]
                rows = k * 16 + iota16
                plsc.store_scatter(
                    dispbuf, [rows, jnp.zeros((16,), jnp.int32)],
                    c16 * dfx[sl])
                plsc.store_scatter(
                    dispbuf, [rows, jnp.full((16,), 1, jnp.int32)],
                    c16 * dfy[sl])
                plsc.store_scatter(
                    dispbuf, [rows, jnp.full((16,), 2, jnp.int32)],
                    c16 * dfz[sl])
            pltpu.sync_copy(mv, msg_acc.at[dstv], add=True)
            pltpu.sync_copy(dispbuf, xacc_sh.at[dstv], add=True)

        return carry

    lax.fori_loop(0, CHUNKS_PER_W, chunk, None)
    plsc.subcore_barrier()

    for i in range(5):
        pltpu.sync_copy(msg_acc.at[pl.ds(r0 + i * 125, 125)], zvm)
        pltpu.sync_copy(zvm, msgp_hbm.at[cidx, pl.ds(r0 + i * 125, 125)])
    pltpu.sync_copy(xacc_sh.at[pl.ds(r0, ROWS_PER_TILE)], zvx)
    pltpu.sync_copy(zvx, xaccp_hbm.at[cidx, pl.ds(r0, ROWS_PER_TILE)])


def _scatter(m, coef, diff, edges, zm, zx):
    f = functools.partial(
        pl.kernel,
        out_type=[jax.ShapeDtypeStruct((NC, N, D), jnp.float32),
                  jax.ShapeDtypeStruct((NC, N, XW), jnp.float32)],
        mesh=_MESH,
        scratch_types=[
            pltpu.VMEM_SHARED((N, D), jnp.float32),
            pltpu.VMEM_SHARED((N, XW), jnp.float32),
            pltpu.VMEM((C,), jnp.int32),
            pltpu.VMEM((C, D), jnp.float32),
            pltpu.VMEM((C,), jnp.float32),
            pltpu.VMEM((C,), jnp.float32),
            pltpu.VMEM((C,), jnp.float32),
            pltpu.VMEM((C,), jnp.float32),
            pltpu.VMEM((C, XW), jnp.float32),
            pltpu.VMEM((125, D), jnp.float32),
            pltpu.VMEM((ROWS_PER_TILE, XW), jnp.float32),
        ],
    )(_scatter_body)
    return f(m, coef, diff, edges, zm, zx)


def _node_body(h_ref, msg0_ref, msg1_ref, xp_ref, xa0_ref, xa1_ref,
               wh1h_ref, wh1m_ref, bh1_ref, wh2_ref, bh2_ref,
               hout_ref, xout_ref):
    h = h_ref[...]
    msg = msg0_ref[...] + msg1_ref[...]
    u = (jnp.dot(h, wh1h_ref[...], preferred_element_type=jnp.float32)
         + jnp.dot(msg, wh1m_ref[...], preferred_element_type=jnp.float32)
         + bh1_ref[...])
    t = u * jax.nn.sigmoid(u)
    hout_ref[...] = (h + jnp.dot(t, wh2_ref[...],
                                 preferred_element_type=jnp.float32)
                     + bh2_ref[...])
    xout_ref[...] = xp_ref[...] + xa0_ref[...] + xa1_ref[...]


def _node(h, msg0, msg1, xp, xa0, xa1, wh1h, wh1m, bh1, wh2, bh2):
    BN = 2000
    grid = N // BN
    full = lambda shape: pl.BlockSpec(shape, lambda i: (0, 0))
    return pl.pallas_call(
        _node_body,
        grid=(grid,),
        in_specs=[
            pl.BlockSpec((BN, D), lambda i: (i, 0)),
            pl.BlockSpec((BN, D), lambda i: (i, 0)),
            pl.BlockSpec((BN, D), lambda i: (i, 0)),
            pl.BlockSpec((BN, XW), lambda i: (i, 0)),
            pl.BlockSpec((BN, XW), lambda i: (i, 0)),
            pl.BlockSpec((BN, XW), lambda i: (i, 0)),
            full((D, D)), full((D, D)), full((1, D)), full((D, D)),
            full((1, D)),
        ],
        out_specs=[
            pl.BlockSpec((BN, D), lambda i: (i, 0)),
            pl.BlockSpec((BN, XW), lambda i: (i, 0)),
        ],
        out_shape=[jax.ShapeDtypeStruct((N, D), jnp.float32),
                   jax.ShapeDtypeStruct((N, XW), jnp.float32)],
    )(h, msg0, msg1, xp, xa0, xa1, wh1h, wh1m, bh1, wh2, bh2)


def kernel(h, x, edges, means, stds, W1, b1, W2, b2, Wa, ba,
           Wx1, bx1, Wx2, Wh1, bh1, Wh2, bh2):
    p, q = _pq(h, W1[:D], W1[D:2 * D])
    ps, qd, diff, d2 = _gather(p, q, x.reshape(-1), edges)
    inv2s2 = 1.0 / (2.0 * stds * stds)
    m, coef = _edge(ps, qd, d2.reshape(E, 1),
                    means.reshape(1, DD), inv2s2.reshape(1, DD),
                    W1[2 * D:], b1.reshape(1, D), W2, b2.reshape(1, D),
                    Wa, ba.reshape(1, 1), Wx1, bx1.reshape(1, D), Wx2)
    zm = jnp.zeros((125, D), jnp.float32)
    zx = jnp.zeros((ROWS_PER_TILE, XW), jnp.float32)
    msgp, xaccp = _scatter(m, coef.reshape(E), diff, edges, zm, zx)
    xpad = jnp.pad(x, ((0, 0), (0, XW - 3)))
    h_new, xsum = _node(h, msgp[0], msgp[1], xpad, xaccp[0], xaccp[1],
                        Wh1[:D], Wh1[D:], bh1.reshape(1, D), Wh2,
                        bh2.reshape(1, D))
    return (h_new, xsum[:, :3])


# speedup vs baseline: 3.0104x; 3.0104x over previous
import functools

import jax
import jax.numpy as jnp
from jax import lax
from jax.experimental import pallas as pl
from jax.experimental.pallas import tpu as pltpu
from jax.experimental.pallas import tpu_sc as plsc

N = 10000
D = 128
DD = 16
E = 320000
R_CUTOFF = 5.0
SPEED = 0.1

NC = 2
NS = 16
NW = NC * NS
C = 128
NCHUNK = E // C
CHUNKS_PER_W = -(-NCHUNK // NW)
CS = 64
NCHUNK_S = E // CS
CHUNKS_S = -(-NCHUNK_S // NW)
BR = 40
NBLK = N // BR
BLKS_PER_TILE = -(-NBLK // NS)

@functools.lru_cache(maxsize=None)
def _sc_mesh():
    return plsc.VectorSubcoreMesh(core_axis_name="c", subcore_axis_name="s",
                                  num_cores=NC, num_subcores=NS)


_SC_PARAMS = pltpu.CompilerParams(needs_layout_passes=False)


def _pq_body(h_ref, w1s_ref, w1d_ref, p_ref, q_ref):
    hb = h_ref[...]
    p_ref[...] = jnp.dot(hb, w1s_ref[...], preferred_element_type=jnp.float32)
    q_ref[...] = jnp.dot(hb, w1d_ref[...], preferred_element_type=jnp.float32)


def _pq(h, w1s, w1d):
    return pl.pallas_call(
        _pq_body,
        out_shape=[jax.ShapeDtypeStruct((N, D), jnp.float32),
                   jax.ShapeDtypeStruct((N, D), jnp.float32)],
    )(h, w1s, w1d)


def _gather_body(p_hbm, q_hbm, xflat_hbm, src_hbm, dst_hbm,
                 ps_hbm, qd_hbm, dbx_hbm, dby_hbm, dbz_hbm, d2_hbm,
                 xtile, srcv, dstv, psbuf, qdbuf, dbx, dby, dbz, d2b,
                 sem1, sem2):
    wid = lax.axis_index("s") * NC + lax.axis_index("c")
    pltpu.sync_copy(xflat_hbm, xtile)

    def chunk(j, carry):
        cid = wid + j * NW

        @pl.when(cid < NCHUNK)
        def _():
            off = cid * C
            pltpu.sync_copy(src_hbm.at[pl.ds(off, C)], srcv)
            pltpu.sync_copy(dst_hbm.at[pl.ds(off, C)], dstv)
            cp1 = pltpu.async_copy(p_hbm.at[srcv], psbuf, sem1)
            cp2 = pltpu.async_copy(q_hbm.at[dstv], qdbuf, sem2)
            for k in range(C // 16):
                sl = pl.ds(k * 16, 16)
                s16 = srcv[sl] * 3
                d16 = dstv[sl] * 3
                dx = (plsc.load_gather(xtile, [d16])
                      - plsc.load_gather(xtile, [s16]))
                dy = (plsc.load_gather(xtile, [d16 + 1])
                      - plsc.load_gather(xtile, [s16 + 1]))
                dz = (plsc.load_gather(xtile, [d16 + 2])
                      - plsc.load_gather(xtile, [s16 + 2]))
                dbx[sl] = dx
                dby[sl] = dy
                dbz[sl] = dz
                d2b[sl] = dx * dx + dy * dy + dz * dz
            cp1.wait()
            cp2.wait()
            pltpu.sync_copy(psbuf, ps_hbm.at[pl.ds(off, C)])
            pltpu.sync_copy(qdbuf, qd_hbm.at[pl.ds(off, C)])
            pltpu.sync_copy(dbx, dbx_hbm.at[pl.ds(off, C)])
            pltpu.sync_copy(dby, dby_hbm.at[pl.ds(off, C)])
            pltpu.sync_copy(dbz, dbz_hbm.at[pl.ds(off, C)])
            pltpu.sync_copy(d2b, d2_hbm.at[pl.ds(off, C)])

        return carry

    lax.fori_loop(0, CHUNKS_S, chunk, None)


def _gather(p, q, xflat, src, dst):
    f = functools.partial(
        pl.kernel,
        out_type=[jax.ShapeDtypeStruct((E, D), jnp.float32),
                  jax.ShapeDtypeStruct((E, D), jnp.float32),
                  jax.ShapeDtypeStruct((E,), jnp.float32),
                  jax.ShapeDtypeStruct((E,), jnp.float32),
                  jax.ShapeDtypeStruct((E,), jnp.float32),
                  jax.ShapeDtypeStruct((E,), jnp.float32)],
        mesh=_sc_mesh(),
        compiler_params=_SC_PARAMS,
        scratch_types=[
            pltpu.VMEM((N * 3,), jnp.float32),
            pltpu.VMEM((C,), jnp.int32),
            pltpu.VMEM((C,), jnp.int32),
            pltpu.VMEM((C, D), jnp.float32),
            pltpu.VMEM((C, D), jnp.float32),
            pltpu.VMEM((C,), jnp.float32),
            pltpu.VMEM((C,), jnp.float32),
            pltpu.VMEM((C,), jnp.float32),
            pltpu.VMEM((C,), jnp.float32),
            pltpu.SemaphoreType.DMA,
            pltpu.SemaphoreType.DMA,
        ],
    )(_gather_body)
    return f(p, q, xflat, src, dst)


BE = 4000


def _edge_body(ps_ref, qd_ref, d2_ref, means_ref, inv2s2_ref,
               w1r_ref, b1_ref, w2_ref, b2_ref, wa_ref, ba_ref,
               wx1_ref, bx1_ref, wx2_ref,
               m_ref, coef_ref):
    d2 = d2_ref[...]
    dist = jnp.sqrt(d2 + 1e-12)
    valid = (dist < R_CUTOFF).astype(jnp.float32)
    delta = dist - means_ref[...]
    rbf = jnp.exp(-delta * delta * inv2s2_ref[...])
    u = (ps_ref[...] + qd_ref[...] + b1_ref[...]
         + jnp.dot(rbf, w1r_ref[...], preferred_element_type=jnp.float32))
    m1 = u * jax.nn.sigmoid(u)
    v = jnp.dot(m1, w2_ref[...], preferred_element_type=jnp.float32) + b2_ref[...]
    m2 = v * jax.nn.sigmoid(v)
    att = jax.nn.sigmoid(
        jnp.dot(m2, wa_ref[...], preferred_element_type=jnp.float32) + ba_ref[...])
    m_att = m2 * att
    m_ref[...] = m_att * valid
    g = jnp.dot(m_att, wx1_ref[...], preferred_element_type=jnp.float32) + bx1_ref[...]
    g = g * jax.nn.sigmoid(g)
    mag = jnp.tanh(jnp.dot(g, wx2_ref[...], preferred_element_type=jnp.float32))
    coef_ref[...] = SPEED * valid * mag / dist


def _edge(ps, qd, d2, means, inv2s2, w1r, b1, w2, b2, wa, ba, wx1, bx1, wx2):
    grid = E // BE
    full = lambda shape: pl.BlockSpec(shape, lambda i: (0, 0))
    return pl.pallas_call(
        _edge_body,
        grid=(grid,),
        in_specs=[
            pl.BlockSpec((BE, D), lambda i: (i, 0)),
            pl.BlockSpec((BE, D), lambda i: (i, 0)),
            pl.BlockSpec((BE, 1), lambda i: (i, 0)),
            full((1, DD)), full((1, DD)),
            full((DD, D)), full((1, D)), full((D, D)), full((1, D)),
            full((D, 1)), full((1, 1)),
            full((D, D)), full((1, D)), full((D, 1)),
        ],
        out_specs=[
            pl.BlockSpec((BE, D), lambda i: (i, 0)),
            pl.BlockSpec((BE, 1), lambda i: (i, 0)),
        ],
        out_shape=[jax.ShapeDtypeStruct((E, D), jnp.float32),
                   jax.ShapeDtypeStruct((E, 1), jnp.float32)],
    )(ps, qd, d2, means, inv2s2, w1r, b1, w2, b2, wa, ba, wx1, bx1, wx2)


XW = 16
NXG = 1280
NXBLK = NXG // BR


def _scatter_body(m_hbm, coef_hbm, dx_hbm, dy_hbm, dz_hbm, dst_hbm, z128_hbm,
                  msgp_hbm, xaccp_hbm,
                  msg_acc, xacc8,
                  dstv, dstg, mv, cv, dfx, dfy, dfz, dispbuf, zvm):
    cidx = lax.axis_index("c")
    sid = lax.axis_index("s")
    wid = sid * NC + cidx

    pltpu.sync_copy(z128_hbm.at[pl.ds(0, BR)], zvm)
    pltpu.sync_copy(z128_hbm, dispbuf)

    def zmsg(j, carry):
        b = sid + j * NS

        @pl.when(b < NBLK)
        def _():
            pltpu.sync_copy(zvm, msg_acc.at[pl.ds(b * BR, BR)])

        return carry

    def zx(j, carry):
        b = sid + j * NS
        pltpu.sync_copy(zvm, xacc8.at[pl.ds(b * BR, BR)])
        return carry

    lax.fori_loop(0, -(-NBLK // NS), zmsg, None)
    lax.fori_loop(0, NXBLK // NS, zx, None)
    plsc.subcore_barrier()

    iota16 = lax.iota(jnp.int32, 16)

    def chunk(j, carry):
        cid = wid + j * NW

        @pl.when(cid < NCHUNK_S)
        def _():
            off = cid * CS
            pltpu.sync_copy(dst_hbm.at[pl.ds(off, CS)], dstv)
            pltpu.sync_copy(m_hbm.at[pl.ds(off, CS)], mv)
            pltpu.sync_copy(coef_hbm.at[pl.ds(off, CS)], cv)
            pltpu.sync_copy(dx_hbm.at[pl.ds(off, CS)], dfx)
            pltpu.sync_copy(dy_hbm.at[pl.ds(off, CS)], dfy)
            pltpu.sync_copy(dz_hbm.at[pl.ds(off, CS)], dfz)
            for k in range(CS // 16):
                sl = pl.ds(k * 16, 16)
                d16 = dstv[sl]
                dstg[sl] = lax.shift_right_logical(d16, 3)
                grp = lax.bitwise_and(d16, 7)
                c16 = cv[sl]
                rows = k * 16 + iota16
                for c, buf in ((0, dfx), (1, dfy), (2, dfz)):
                    val = c16 * buf[sl]
                    for g in range(8):
                        plsc.store_scatter(
                            dispbuf, [rows, jnp.full((16,), g * 16 + c,
                                                     jnp.int32)],
                            jnp.where(grp == g, val, 0.0))
            pltpu.sync_copy(mv, msg_acc.at[dstv], add=True)
            pltpu.sync_copy(dispbuf, xacc8.at[dstg], add=True)

        return carry

    lax.fori_loop(0, CHUNKS_S, chunk, None)
    plsc.subcore_barrier()

    def pmsg(j, carry):
        b = sid + j * NS

        @pl.when(b < NBLK)
        def _():
            pltpu.sync_copy(msg_acc.at[pl.ds(b * BR, BR)], zvm)
            pltpu.sync_copy(zvm, msgp_hbm.at[pl.ds(cidx * N + b * BR, BR)])

        return carry

    def px(j, carry):
        b = sid + j * NS
        pltpu.sync_copy(xacc8.at[pl.ds(b * BR, BR)], zvm)
        pltpu.sync_copy(zvm, xaccp_hbm.at[pl.ds(cidx * NXG + b * BR, BR)])
        return carry

    lax.fori_loop(0, -(-NBLK // NS), pmsg, None)
    lax.fori_loop(0, NXBLK // NS, px, None)


def _scatter(m, coef, dx, dy, dz, dst, z128):
    f = functools.partial(
        pl.kernel,
        out_type=[jax.ShapeDtypeStruct((NC * N, D), jnp.float32),
                  jax.ShapeDtypeStruct((NC * NXG, D), jnp.float32)],
        mesh=_sc_mesh(),
        compiler_params=_SC_PARAMS,
        scratch_types=[
            pltpu.VMEM_SHARED((N, D), jnp.float32),
            pltpu.VMEM_SHARED((NXG, D), jnp.float32),
            pltpu.VMEM((CS,), jnp.int32),
            pltpu.VMEM((CS,), jnp.int32),
            pltpu.VMEM((CS, D), jnp.float32),
            pltpu.VMEM((CS,), jnp.float32),
            pltpu.VMEM((CS,), jnp.float32),
            pltpu.VMEM((CS,), jnp.float32),
            pltpu.VMEM((CS,), jnp.float32),
            pltpu.VMEM((CS, D), jnp.float32),
            pltpu.VMEM((BR, D), jnp.float32),
        ],
    )(_scatter_body)
    return f(m, coef, dx, dy, dz, dst, z128)


def _node_body(h_ref, msg0_ref, msg1_ref, xp_ref, xa0_ref, xa1_ref,
               wh1h_ref, wh1m_ref, bh1_ref, wh2_ref, bh2_ref,
               hout_ref, xout_ref):
    h = h_ref[...]
    msg = msg0_ref[...] + msg1_ref[...]
    u = (jnp.dot(h, wh1h_ref[...], preferred_element_type=jnp.float32)
         + jnp.dot(msg, wh1m_ref[...], preferred_element_type=jnp.float32)
         + bh1_ref[...])
    t = u * jax.nn.sigmoid(u)
    hout_ref[...] = (h + jnp.dot(t, wh2_ref[...],
                                 preferred_element_type=jnp.float32)
                     + bh2_ref[...])
    xout_ref[...] = xp_ref[...] + xa0_ref[...] + xa1_ref[...]


def _node(h, msg0, msg1, xp, xa0, xa1, wh1h, wh1m, bh1, wh2, bh2):
    BN = 2000
    grid = N // BN
    full = lambda shape: pl.BlockSpec(shape, lambda i: (0, 0))
    return pl.pallas_call(
        _node_body,
        grid=(grid,),
        in_specs=[
            pl.BlockSpec((BN, D), lambda i: (i, 0)),
            pl.BlockSpec((BN, D), lambda i: (i, 0)),
            pl.BlockSpec((BN, D), lambda i: (i, 0)),
            pl.BlockSpec((BN, XW), lambda i: (i, 0)),
            pl.BlockSpec((BN, XW), lambda i: (i, 0)),
            pl.BlockSpec((BN, XW), lambda i: (i, 0)),
            full((D, D)), full((D, D)), full((1, D)), full((D, D)),
            full((1, D)),
        ],
        out_specs=[
            pl.BlockSpec((BN, D), lambda i: (i, 0)),
            pl.BlockSpec((BN, XW), lambda i: (i, 0)),
        ],
        out_shape=[jax.ShapeDtypeStruct((N, D), jnp.float32),
                   jax.ShapeDtypeStruct((N, XW), jnp.float32)],
    )(h, msg0, msg1, xp, xa0, xa1, wh1h, wh1m, bh1, wh2, bh2)


def kernel(h, x, edges, means, stds, W1, b1, W2, b2, Wa, ba,
           Wx1, bx1, Wx2, Wh1, bh1, Wh2, bh2):
    p, q = _pq(h, W1[:D], W1[D:2 * D])
    src_idx = edges[0]
    dst_idx = edges[1]
    ps, qd, dbx, dby, dbz, d2 = _gather(p, q, x.reshape(-1), src_idx, dst_idx)
    inv2s2 = 1.0 / (2.0 * stds * stds)
    m, coef = _edge(ps, qd, d2.reshape(E, 1),
                    means.reshape(1, DD), inv2s2.reshape(1, DD),
                    W1[2 * D:], b1.reshape(1, D), W2, b2.reshape(1, D),
                    Wa, ba.reshape(1, 1), Wx1, bx1.reshape(1, D), Wx2)
    z128 = jnp.zeros((CS, D), jnp.float32)
    msgp, xaccp = _scatter(m, coef.reshape(E), dbx, dby, dbz, dst_idx, z128)
    msgp = msgp.reshape(NC, N, D)
    xaccp = xaccp.reshape(NC, NXG, D)[:, :N // 8].reshape(NC, N, XW)
    xpad = jnp.pad(x, ((0, 0), (0, XW - 3)))
    h_new, xsum = _node(h, msgp[0], msgp[1], xpad, xaccp[0], xaccp[1],
                        Wh1[:D], Wh1[D:], bh1.reshape(1, D), Wh2,
                        bh2.reshape(1, D))
    return (h_new, xsum[:, :3])


# speedup vs baseline: 3.9413x; 1.3092x over previous
import functools

import jax
import jax.numpy as jnp
from jax import lax
from jax.experimental import pallas as pl
from jax.experimental.pallas import tpu as pltpu
from jax.experimental.pallas import tpu_sc as plsc

N = 10000
D = 128
DD = 16
E = 320000
R_CUTOFF = 5.0
SPEED = 0.1

NC = 2
NS = 16
NW = NC * NS
C = 128
NCHUNK = E // C
CHUNKS_PER_W = -(-NCHUNK // NW)
CS = 64
NCHUNK_S = E // CS
CHUNKS_S = -(-NCHUNK_S // NW)
BR = 40
NBLK = N // BR
BLKS_PER_TILE = -(-NBLK // NS)

@functools.lru_cache(maxsize=None)
def _sc_mesh():
    return plsc.VectorSubcoreMesh(core_axis_name="c", subcore_axis_name="s",
                                  num_cores=NC, num_subcores=NS)


_SC_PARAMS = pltpu.CompilerParams(needs_layout_passes=False)


def _pq_body(h_ref, w1s_ref, w1d_ref, p_ref, q_ref):
    hb = h_ref[...]
    p_ref[...] = jnp.dot(hb, w1s_ref[...], preferred_element_type=jnp.float32)
    q_ref[...] = jnp.dot(hb, w1d_ref[...], preferred_element_type=jnp.float32)


def _pq(h, w1s, w1d):
    return pl.pallas_call(
        _pq_body,
        out_shape=[jax.ShapeDtypeStruct((N, D), jnp.float32),
                   jax.ShapeDtypeStruct((N, D), jnp.float32)],
    )(h, w1s, w1d)


def _gather_body(p_hbm, q_hbm, xflat_hbm, src_hbm, dst_hbm,
                 ps_hbm, qd_hbm, dbx_hbm, dby_hbm, dbz_hbm, d2_hbm,
                 xtile, srcv, dstv, psbuf, qdbuf, dbx, dby, dbz, d2b,
                 sem1, sem2, wsem):
    wid = lax.axis_index("s") * NC + lax.axis_index("c")
    pltpu.sync_copy(xflat_hbm, xtile)

    def wdrain(off):
        pltpu.make_async_copy(psbuf, ps_hbm.at[pl.ds(off, C)], wsem).wait()
        pltpu.make_async_copy(qdbuf, qd_hbm.at[pl.ds(off, C)], wsem).wait()
        pltpu.make_async_copy(dbx, dbx_hbm.at[pl.ds(off, C)], wsem).wait()
        pltpu.make_async_copy(dby, dby_hbm.at[pl.ds(off, C)], wsem).wait()
        pltpu.make_async_copy(dbz, dbz_hbm.at[pl.ds(off, C)], wsem).wait()
        pltpu.make_async_copy(d2b, d2_hbm.at[pl.ds(off, C)], wsem).wait()

    def chunk(j, carry):
        cid = wid + j * NW

        @pl.when(cid < NCHUNK)
        def _():
            off = cid * C

            @pl.when(j > 0)
            def _():
                wdrain(off)

            ld1 = pltpu.async_copy(src_hbm.at[pl.ds(off, C)], srcv, sem1)
            ld2 = pltpu.async_copy(dst_hbm.at[pl.ds(off, C)], dstv, sem2)
            ld1.wait()
            ld2.wait()
            cp1 = pltpu.async_copy(p_hbm.at[srcv], psbuf, sem1)
            cp2 = pltpu.async_copy(q_hbm.at[dstv], qdbuf, sem2)
            for k in range(C // 16):
                sl = pl.ds(k * 16, 16)
                s16 = srcv[sl] * 3
                d16 = dstv[sl] * 3
                dx = (plsc.load_gather(xtile, [d16])
                      - plsc.load_gather(xtile, [s16]))
                dy = (plsc.load_gather(xtile, [d16 + 1])
                      - plsc.load_gather(xtile, [s16 + 1]))
                dz = (plsc.load_gather(xtile, [d16 + 2])
                      - plsc.load_gather(xtile, [s16 + 2]))
                dbx[sl] = dx
                dby[sl] = dy
                dbz[sl] = dz
                d2b[sl] = dx * dx + dy * dy + dz * dz
            cp1.wait()
            cp2.wait()
            pltpu.async_copy(psbuf, ps_hbm.at[pl.ds(off, C)], wsem)
            pltpu.async_copy(qdbuf, qd_hbm.at[pl.ds(off, C)], wsem)
            pltpu.async_copy(dbx, dbx_hbm.at[pl.ds(off, C)], wsem)
            pltpu.async_copy(dby, dby_hbm.at[pl.ds(off, C)], wsem)
            pltpu.async_copy(dbz, dbz_hbm.at[pl.ds(off, C)], wsem)
            pltpu.async_copy(d2b, d2_hbm.at[pl.ds(off, C)], wsem)

        return carry

    lax.fori_loop(0, CHUNKS_PER_W, chunk, None)
    wdrain(wid * C)


def _gather(p, q, xflat, src, dst):
    f = functools.partial(
        pl.kernel,
        out_type=[jax.ShapeDtypeStruct((E, D), jnp.float32),
                  jax.ShapeDtypeStruct((E, D), jnp.float32),
                  jax.ShapeDtypeStruct((E,), jnp.float32),
                  jax.ShapeDtypeStruct((E,), jnp.float32),
                  jax.ShapeDtypeStruct((E,), jnp.float32),
                  jax.ShapeDtypeStruct((E,), jnp.float32)],
        mesh=_sc_mesh(),
        compiler_params=_SC_PARAMS,
        scratch_types=[
            pltpu.VMEM((N * 3,), jnp.float32),
            pltpu.VMEM((C,), jnp.int32),
            pltpu.VMEM((C,), jnp.int32),
            pltpu.VMEM((C, D), jnp.float32),
            pltpu.VMEM((C, D), jnp.float32),
            pltpu.VMEM((C,), jnp.float32),
            pltpu.VMEM((C,), jnp.float32),
            pltpu.VMEM((C,), jnp.float32),
            pltpu.VMEM((C,), jnp.float32),
            pltpu.SemaphoreType.DMA,
            pltpu.SemaphoreType.DMA,
            pltpu.SemaphoreType.DMA,
        ],
    )(_gather_body)
    return f(p, q, xflat, src, dst)


BE = 4000


def _edge_body(ps_ref, qd_ref, d2_ref, means_ref, inv2s2_ref,
               w1r_ref, b1_ref, w2_ref, b2_ref, wa_ref, ba_ref,
               wx1_ref, bx1_ref, wx2_ref,
               m_ref, coef_ref):
    d2 = d2_ref[...]
    dist = jnp.sqrt(d2 + 1e-12)
    valid = (dist < R_CUTOFF).astype(jnp.float32)
    delta = dist - means_ref[...]
    rbf = jnp.exp(-delta * delta * inv2s2_ref[...])
    u = (ps_ref[...] + qd_ref[...] + b1_ref[...]
         + jnp.dot(rbf, w1r_ref[...], preferred_element_type=jnp.float32))
    m1 = u * jax.nn.sigmoid(u)
    v = jnp.dot(m1, w2_ref[...], preferred_element_type=jnp.float32) + b2_ref[...]
    m2 = v * jax.nn.sigmoid(v)
    att = jax.nn.sigmoid(
        jnp.dot(m2, wa_ref[...], preferred_element_type=jnp.float32) + ba_ref[...])
    m_att = m2 * att
    m_ref[...] = m_att * valid
    g = jnp.dot(m_att, wx1_ref[...], preferred_element_type=jnp.float32) + bx1_ref[...]
    g = g * jax.nn.sigmoid(g)
    mag = jnp.tanh(jnp.dot(g, wx2_ref[...], preferred_element_type=jnp.float32))
    coef_ref[...] = SPEED * valid * mag / dist


def _edge(ps, qd, d2, means, inv2s2, w1r, b1, w2, b2, wa, ba, wx1, bx1, wx2):
    grid = E // BE
    full = lambda shape: pl.BlockSpec(shape, lambda i: (0, 0))
    return pl.pallas_call(
        _edge_body,
        grid=(grid,),
        in_specs=[
            pl.BlockSpec((BE, D), lambda i: (i, 0)),
            pl.BlockSpec((BE, D), lambda i: (i, 0)),
            pl.BlockSpec((BE, 1), lambda i: (i, 0)),
            full((1, DD)), full((1, DD)),
            full((DD, D)), full((1, D)), full((D, D)), full((1, D)),
            full((D, 1)), full((1, 1)),
            full((D, D)), full((1, D)), full((D, 1)),
        ],
        out_specs=[
            pl.BlockSpec((BE, D), lambda i: (i, 0)),
            pl.BlockSpec((BE, 1), lambda i: (i, 0)),
        ],
        out_shape=[jax.ShapeDtypeStruct((E, D), jnp.float32),
                   jax.ShapeDtypeStruct((E, 1), jnp.float32)],
    )(ps, qd, d2, means, inv2s2, w1r, b1, w2, b2, wa, ba, wx1, bx1, wx2)


XW = 16
NXG = 1280
NXBLK = NXG // BR


def _scatter_body(m_hbm, coef_hbm, dx_hbm, dy_hbm, dz_hbm, dst_hbm, z128_hbm,
                  msgp_hbm, xaccp_hbm,
                  msg_acc, xacc8,
                  dstv, dstg, mv, cv, dfx, dfy, dfz, dispbuf, zvm,
                  lsem, asem):
    cidx = lax.axis_index("c")
    sid = lax.axis_index("s")
    wid = sid * NC + cidx

    pltpu.sync_copy(z128_hbm.at[pl.ds(0, BR)], zvm)
    pltpu.sync_copy(z128_hbm, dispbuf)

    def zmsg(j, carry):
        b = sid + j * NS

        @pl.when(b < NBLK)
        def _():
            pltpu.sync_copy(zvm, msg_acc.at[pl.ds(b * BR, BR)])

        return carry

    def zx(j, carry):
        b = sid + j * NS
        pltpu.sync_copy(zvm, xacc8.at[pl.ds(b * BR, BR)])
        return carry

    lax.fori_loop(0, -(-NBLK // NS), zmsg, None)
    lax.fori_loop(0, NXBLK // NS, zx, None)
    plsc.subcore_barrier()

    iota16 = lax.iota(jnp.int32, 16)

    def adrain():
        pltpu.make_async_copy(mv, msg_acc.at[dstv], asem).wait()
        pltpu.make_async_copy(dispbuf, xacc8.at[dstg], asem).wait()

    def chunk(j, carry):
        cid = wid + j * NW

        @pl.when(cid < NCHUNK_S)
        def _():
            off = cid * CS

            @pl.when(j > 0)
            def _():
                adrain()

            l1 = pltpu.async_copy(dst_hbm.at[pl.ds(off, CS)], dstv, lsem)
            l2 = pltpu.async_copy(m_hbm.at[pl.ds(off, CS)], mv, lsem)
            l3 = pltpu.async_copy(coef_hbm.at[pl.ds(off, CS)], cv, lsem)
            l4 = pltpu.async_copy(dx_hbm.at[pl.ds(off, CS)], dfx, lsem)
            l5 = pltpu.async_copy(dy_hbm.at[pl.ds(off, CS)], dfy, lsem)
            l6 = pltpu.async_copy(dz_hbm.at[pl.ds(off, CS)], dfz, lsem)
            l1.wait()
            l2.wait()
            l3.wait()
            l4.wait()
            l5.wait()
            l6.wait()
            for k in range(CS // 16):
                sl = pl.ds(k * 16, 16)
                d16 = dstv[sl]
                dstg[sl] = lax.shift_right_logical(d16, 3)
                grp = lax.bitwise_and(d16, 7)
                c16 = cv[sl]
                rows = k * 16 + iota16
                for c, buf in ((0, dfx), (1, dfy), (2, dfz)):
                    val = c16 * buf[sl]
                    for g in range(8):
                        plsc.store_scatter(
                            dispbuf, [rows, jnp.full((16,), g * 16 + c,
                                                     jnp.int32)],
                            jnp.where(grp == g, val, 0.0))
            pltpu.async_copy(mv, msg_acc.at[dstv], asem, add=True)
            pltpu.async_copy(dispbuf, xacc8.at[dstg], asem, add=True)

        return carry

    lax.fori_loop(0, CHUNKS_S, chunk, None)
    adrain()
    plsc.subcore_barrier()

    def pmsg(j, carry):
        b = sid + j * NS

        @pl.when(b < NBLK)
        def _():
            pltpu.sync_copy(msg_acc.at[pl.ds(b * BR, BR)], zvm)
            pltpu.sync_copy(zvm, msgp_hbm.at[pl.ds(cidx * N + b * BR, BR)])

        return carry

    def px(j, carry):
        b = sid + j * NS
        pltpu.sync_copy(xacc8.at[pl.ds(b * BR, BR)], zvm)
        pltpu.sync_copy(zvm, xaccp_hbm.at[pl.ds(cidx * NXG + b * BR, BR)])
        return carry

    lax.fori_loop(0, -(-NBLK // NS), pmsg, None)
    lax.fori_loop(0, NXBLK // NS, px, None)


def _scatter(m, coef, dx, dy, dz, dst, z128):
    f = functools.partial(
        pl.kernel,
        out_type=[jax.ShapeDtypeStruct((NC * N, D), jnp.float32),
                  jax.ShapeDtypeStruct((NC * NXG, D), jnp.float32)],
        mesh=_sc_mesh(),
        compiler_params=_SC_PARAMS,
        scratch_types=[
            pltpu.VMEM_SHARED((N, D), jnp.float32),
            pltpu.VMEM_SHARED((NXG, D), jnp.float32),
            pltpu.VMEM((CS,), jnp.int32),
            pltpu.VMEM((CS,), jnp.int32),
            pltpu.VMEM((CS, D), jnp.float32),
            pltpu.VMEM((CS,), jnp.float32),
            pltpu.VMEM((CS,), jnp.float32),
            pltpu.VMEM((CS,), jnp.float32),
            pltpu.VMEM((CS,), jnp.float32),
            pltpu.VMEM((CS, D), jnp.float32),
            pltpu.VMEM((BR, D), jnp.float32),
            pltpu.SemaphoreType.DMA,
            pltpu.SemaphoreType.DMA,
        ],
    )(_scatter_body)
    return f(m, coef, dx, dy, dz, dst, z128)


def _node_body(h_ref, msg0_ref, msg1_ref, xp_ref, xa0_ref, xa1_ref,
               wh1h_ref, wh1m_ref, bh1_ref, wh2_ref, bh2_ref,
               hout_ref, xout_ref):
    h = h_ref[...]
    msg = msg0_ref[...] + msg1_ref[...]
    u = (jnp.dot(h, wh1h_ref[...], preferred_element_type=jnp.float32)
         + jnp.dot(msg, wh1m_ref[...], preferred_element_type=jnp.float32)
         + bh1_ref[...])
    t = u * jax.nn.sigmoid(u)
    hout_ref[...] = (h + jnp.dot(t, wh2_ref[...],
                                 preferred_element_type=jnp.float32)
                     + bh2_ref[...])
    xout_ref[...] = xp_ref[...] + xa0_ref[...] + xa1_ref[...]


def _node(h, msg0, msg1, xp, xa0, xa1, wh1h, wh1m, bh1, wh2, bh2):
    BN = 2000
    grid = N // BN
    full = lambda shape: pl.BlockSpec(shape, lambda i: (0, 0))
    return pl.pallas_call(
        _node_body,
        grid=(grid,),
        in_specs=[
            pl.BlockSpec((BN, D), lambda i: (i, 0)),
            pl.BlockSpec((BN, D), lambda i: (i, 0)),
            pl.BlockSpec((BN, D), lambda i: (i, 0)),
            pl.BlockSpec((BN, XW), lambda i: (i, 0)),
            pl.BlockSpec((BN, XW), lambda i: (i, 0)),
            pl.BlockSpec((BN, XW), lambda i: (i, 0)),
            full((D, D)), full((D, D)), full((1, D)), full((D, D)),
            full((1, D)),
        ],
        out_specs=[
            pl.BlockSpec((BN, D), lambda i: (i, 0)),
            pl.BlockSpec((BN, XW), lambda i: (i, 0)),
        ],
        out_shape=[jax.ShapeDtypeStruct((N, D), jnp.float32),
                   jax.ShapeDtypeStruct((N, XW), jnp.float32)],
    )(h, msg0, msg1, xp, xa0, xa1, wh1h, wh1m, bh1, wh2, bh2)


def kernel(h, x, edges, means, stds, W1, b1, W2, b2, Wa, ba,
           Wx1, bx1, Wx2, Wh1, bh1, Wh2, bh2):
    p, q = _pq(h, W1[:D], W1[D:2 * D])
    src_idx = edges[0]
    dst_idx = edges[1]
    ps, qd, dbx, dby, dbz, d2 = _gather(p, q, x.reshape(-1), src_idx, dst_idx)
    inv2s2 = 1.0 / (2.0 * stds * stds)
    m, coef = _edge(ps, qd, d2.reshape(E, 1),
                    means.reshape(1, DD), inv2s2.reshape(1, DD),
                    W1[2 * D:], b1.reshape(1, D), W2, b2.reshape(1, D),
                    Wa, ba.reshape(1, 1), Wx1, bx1.reshape(1, D), Wx2)
    z128 = jnp.zeros((CS, D), jnp.float32)
    msgp, xaccp = _scatter(m, coef.reshape(E), dbx, dby, dbz, dst_idx, z128)
    msgp = msgp.reshape(NC, N, D)
    xaccp = xaccp.reshape(NC, NXG, D)[:, :N // 8].reshape(NC, N, XW)
    xpad = jnp.pad(x, ((0, 0), (0, XW - 3)))
    h_new, xsum = _node(h, msgp[0], msgp[1], xpad, xaccp[0], xaccp[1],
                        Wh1[:D], Wh1[D:], bh1.reshape(1, D), Wh2,
                        bh2.reshape(1, D))
    return (h_new, xsum[:, :3])


# speedup vs baseline: 4.5593x; 1.1568x over previous
import functools

import jax
import jax.numpy as jnp
from jax import lax
from jax.experimental import pallas as pl
from jax.experimental.pallas import tpu as pltpu
from jax.experimental.pallas import tpu_sc as plsc

N = 10000
D = 128
DD = 16
E = 320000
R_CUTOFF = 5.0
SPEED = 0.1

NC = 2
NS = 16
NW = NC * NS
C = 128
NCHUNK = E // C
CHUNKS_PER_W = -(-NCHUNK // NW)
CS = 32
NCHUNK_S = E // CS
CHUNKS_S = -(-NCHUNK_S // NW)
BR = 40
NBLK = N // BR
BLKS_PER_TILE = -(-NBLK // NS)

@functools.lru_cache(maxsize=None)
def _sc_mesh():
    return plsc.VectorSubcoreMesh(core_axis_name="c", subcore_axis_name="s",
                                  num_cores=NC, num_subcores=NS)


_SC_PARAMS = pltpu.CompilerParams(needs_layout_passes=False)


def _pq_body(h_ref, w1s_ref, w1d_ref, p_ref, q_ref):
    hb = h_ref[...]
    p_ref[...] = jnp.dot(hb, w1s_ref[...], preferred_element_type=jnp.float32)
    q_ref[...] = jnp.dot(hb, w1d_ref[...], preferred_element_type=jnp.float32)


def _pq(h, w1s, w1d):
    return pl.pallas_call(
        _pq_body,
        out_shape=[jax.ShapeDtypeStruct((N, D), jnp.float32),
                   jax.ShapeDtypeStruct((N, D), jnp.float32)],
    )(h, w1s, w1d)


def _gather_body(p_hbm, q_hbm, xflat_hbm, src_hbm, dst_hbm,
                 ps_hbm, qd_hbm, dbx_hbm, dby_hbm, dbz_hbm, d2_hbm,
                 xtile, srcv, dstv, psbuf, qdbuf, dbx, dby, dbz, d2b,
                 sem1, sem2, wsem):
    wid = lax.axis_index("s") * NC + lax.axis_index("c")
    pltpu.sync_copy(xflat_hbm, xtile)

    def wdrain(off):
        pltpu.make_async_copy(psbuf, ps_hbm.at[pl.ds(off, C)], wsem).wait()
        pltpu.make_async_copy(qdbuf, qd_hbm.at[pl.ds(off, C)], wsem).wait()
        pltpu.make_async_copy(dbx, dbx_hbm.at[pl.ds(off, C)], wsem).wait()
        pltpu.make_async_copy(dby, dby_hbm.at[pl.ds(off, C)], wsem).wait()
        pltpu.make_async_copy(dbz, dbz_hbm.at[pl.ds(off, C)], wsem).wait()
        pltpu.make_async_copy(d2b, d2_hbm.at[pl.ds(off, C)], wsem).wait()

    def chunk(j, carry):
        cid = wid + j * NW

        @pl.when(cid < NCHUNK)
        def _():
            off = cid * C

            @pl.when(j > 0)
            def _():
                wdrain(off)

            ld1 = pltpu.async_copy(src_hbm.at[pl.ds(off, C)], srcv, sem1)
            ld2 = pltpu.async_copy(dst_hbm.at[pl.ds(off, C)], dstv, sem2)
            ld1.wait()
            ld2.wait()
            cp1 = pltpu.async_copy(p_hbm.at[srcv], psbuf, sem1)
            cp2 = pltpu.async_copy(q_hbm.at[dstv], qdbuf, sem2)
            for k in range(C // 16):
                sl = pl.ds(k * 16, 16)
                s16 = srcv[sl] * 3
                d16 = dstv[sl] * 3
                dx = (plsc.load_gather(xtile, [d16])
                      - plsc.load_gather(xtile, [s16]))
                dy = (plsc.load_gather(xtile, [d16 + 1])
                      - plsc.load_gather(xtile, [s16 + 1]))
                dz = (plsc.load_gather(xtile, [d16 + 2])
                      - plsc.load_gather(xtile, [s16 + 2]))
                dbx[sl] = dx
                dby[sl] = dy
                dbz[sl] = dz
                d2b[sl] = dx * dx + dy * dy + dz * dz
            cp1.wait()
            cp2.wait()
            pltpu.async_copy(psbuf, ps_hbm.at[pl.ds(off, C)], wsem)
            pltpu.async_copy(qdbuf, qd_hbm.at[pl.ds(off, C)], wsem)
            pltpu.async_copy(dbx, dbx_hbm.at[pl.ds(off, C)], wsem)
            pltpu.async_copy(dby, dby_hbm.at[pl.ds(off, C)], wsem)
            pltpu.async_copy(dbz, dbz_hbm.at[pl.ds(off, C)], wsem)
            pltpu.async_copy(d2b, d2_hbm.at[pl.ds(off, C)], wsem)

        return carry

    lax.fori_loop(0, CHUNKS_PER_W, chunk, None)
    wdrain(wid * C)


def _gather(p, q, xflat, src, dst):
    f = functools.partial(
        pl.kernel,
        out_type=[jax.ShapeDtypeStruct((E, D), jnp.float32),
                  jax.ShapeDtypeStruct((E, D), jnp.float32),
                  jax.ShapeDtypeStruct((E,), jnp.float32),
                  jax.ShapeDtypeStruct((E,), jnp.float32),
                  jax.ShapeDtypeStruct((E,), jnp.float32),
                  jax.ShapeDtypeStruct((E,), jnp.float32)],
        mesh=_sc_mesh(),
        compiler_params=_SC_PARAMS,
        scratch_types=[
            pltpu.VMEM((N * 3,), jnp.float32),
            pltpu.VMEM((C,), jnp.int32),
            pltpu.VMEM((C,), jnp.int32),
            pltpu.VMEM((C, D), jnp.float32),
            pltpu.VMEM((C, D), jnp.float32),
            pltpu.VMEM((C,), jnp.float32),
            pltpu.VMEM((C,), jnp.float32),
            pltpu.VMEM((C,), jnp.float32),
            pltpu.VMEM((C,), jnp.float32),
            pltpu.SemaphoreType.DMA,
            pltpu.SemaphoreType.DMA,
            pltpu.SemaphoreType.DMA,
        ],
    )(_gather_body)
    return f(p, q, xflat, src, dst)


BE = 4000


def _edge_body(ps_ref, qd_ref, d2_ref, means_ref, inv2s2_ref,
               w1r_ref, b1_ref, w2_ref, b2_ref, wa_ref, ba_ref,
               wx1_ref, bx1_ref, wx2_ref,
               m_ref, coef_ref):
    d2 = d2_ref[...]
    dist = jnp.sqrt(d2 + 1e-12)
    valid = (dist < R_CUTOFF).astype(jnp.float32)
    delta = dist - means_ref[...]
    rbf = jnp.exp(-delta * delta * inv2s2_ref[...])
    u = (ps_ref[...] + qd_ref[...] + b1_ref[...]
         + jnp.dot(rbf, w1r_ref[...], preferred_element_type=jnp.float32))
    m1 = u * jax.nn.sigmoid(u)
    v = jnp.dot(m1, w2_ref[...], preferred_element_type=jnp.float32) + b2_ref[...]
    m2 = v * jax.nn.sigmoid(v)
    att = jax.nn.sigmoid(
        jnp.dot(m2, wa_ref[...], preferred_element_type=jnp.float32) + ba_ref[...])
    m_att = m2 * att
    m_ref[...] = m_att * valid
    g = jnp.dot(m_att, wx1_ref[...], preferred_element_type=jnp.float32) + bx1_ref[...]
    g = g * jax.nn.sigmoid(g)
    mag = jnp.tanh(jnp.dot(g, wx2_ref[...], preferred_element_type=jnp.float32))
    coef_ref[...] = SPEED * valid * mag / dist


def _edge(ps, qd, d2, means, inv2s2, w1r, b1, w2, b2, wa, ba, wx1, bx1, wx2):
    grid = E // BE
    full = lambda shape: pl.BlockSpec(shape, lambda i: (0, 0))
    return pl.pallas_call(
        _edge_body,
        grid=(grid,),
        in_specs=[
            pl.BlockSpec((BE, D), lambda i: (i, 0)),
            pl.BlockSpec((BE, D), lambda i: (i, 0)),
            pl.BlockSpec((BE, 1), lambda i: (i, 0)),
            full((1, DD)), full((1, DD)),
            full((DD, D)), full((1, D)), full((D, D)), full((1, D)),
            full((D, 1)), full((1, 1)),
            full((D, D)), full((1, D)), full((D, 1)),
        ],
        out_specs=[
            pl.BlockSpec((BE, D), lambda i: (i, 0)),
            pl.BlockSpec((BE, 1), lambda i: (i, 0)),
        ],
        out_shape=[jax.ShapeDtypeStruct((E, D), jnp.float32),
                   jax.ShapeDtypeStruct((E, 1), jnp.float32)],
    )(ps, qd, d2, means, inv2s2, w1r, b1, w2, b2, wa, ba, wx1, bx1, wx2)


XW = 16
NBX = N * XW // 640


def _scatter_body(m_hbm, coef_hbm, dx_hbm, dy_hbm, dz_hbm, dst_hbm,
                  zm_hbm, zf_hbm,
                  msgp_hbm, xaccp_hbm,
                  msg_acc, xacc_flat,
                  dstv0, mv0, cv0, dfx0, dfy0, dfz0, dsp0, dsi0,
                  dstv1, mv1, cv1, dfx1, dfy1, dfz1, dsp1, dsi1,
                  zvm, zbf,
                  lsem0, lsem1, asem0, asem1):
    cidx = lax.axis_index("c")
    sid = lax.axis_index("s")
    wid = sid * NC + cidx

    sets = ((dstv0, mv0, cv0, dfx0, dfy0, dfz0, dsp0, dsi0, lsem0, asem0),
            (dstv1, mv1, cv1, dfx1, dfy1, dfz1, dsp1, dsi1, lsem1, asem1))

    pltpu.sync_copy(zm_hbm, zvm)
    pltpu.sync_copy(zf_hbm, zbf)

    def zblk(j, carry):
        b = sid + j * NS

        @pl.when(b < NBLK)
        def _():
            pltpu.sync_copy(zvm, msg_acc.at[pl.ds(b * BR, BR)])
            pltpu.sync_copy(zbf, xacc_flat.at[pl.ds(b * 640, 640)])

        return carry

    lax.fori_loop(0, -(-NBLK // NS), zblk, None)
    plsc.subcore_barrier()

    iota16 = lax.iota(jnp.int32, 16)

    def issue_loads(p, off):
        (dstv, mv, cv, dfx, dfy, dfz, dsp, dsi, lsem, asem) = sets[p]
        pltpu.async_copy(dst_hbm.at[pl.ds(off, CS)], dstv, lsem)
        pltpu.async_copy(m_hbm.at[pl.ds(off, CS)], mv, lsem)
        pltpu.async_copy(coef_hbm.at[pl.ds(off, CS)], cv, lsem)
        pltpu.async_copy(dx_hbm.at[pl.ds(off, CS)], dfx, lsem)
        pltpu.async_copy(dy_hbm.at[pl.ds(off, CS)], dfy, lsem)
        pltpu.async_copy(dz_hbm.at[pl.ds(off, CS)], dfz, lsem)

    def ldrain(p, off):
        (dstv, mv, cv, dfx, dfy, dfz, dsp, dsi, lsem, asem) = sets[p]
        pltpu.make_async_copy(dst_hbm.at[pl.ds(off, CS)], dstv, lsem).wait()
        pltpu.make_async_copy(m_hbm.at[pl.ds(off, CS)], mv, lsem).wait()
        pltpu.make_async_copy(coef_hbm.at[pl.ds(off, CS)], cv, lsem).wait()
        pltpu.make_async_copy(dx_hbm.at[pl.ds(off, CS)], dfx, lsem).wait()
        pltpu.make_async_copy(dy_hbm.at[pl.ds(off, CS)], dfy, lsem).wait()
        pltpu.make_async_copy(dz_hbm.at[pl.ds(off, CS)], dfz, lsem).wait()

    def build_and_add(p):
        (dstv, mv, cv, dfx, dfy, dfz, dsp, dsi, lsem, asem) = sets[p]
        for k in range(CS // 16):
            sl = pl.ds(k * 16, 16)
            d16 = dstv[sl]
            c16 = cv[sl]
            rows3 = (k * 16 + iota16) * 3
            for c, buf in ((0, dfx), (1, dfy), (2, dfz)):
                plsc.store_scatter(dsi, [rows3 + c], d16 * XW + c)
                plsc.store_scatter(dsp, [rows3 + c], c16 * buf[sl])
        pltpu.async_copy(mv, msg_acc.at[dstv], asem, add=True)
        pltpu.async_copy(dsp, xacc_flat.at[dsi], asem, add=True)

    def adrain(p):
        (dstv, mv, cv, dfx, dfy, dfz, dsp, dsi, lsem, asem) = sets[p]
        pltpu.make_async_copy(mv, msg_acc.at[dstv], asem).wait()
        pltpu.make_async_copy(dsp, xacc_flat.at[dsi], asem).wait()

    issue_loads(0, wid * CS)

    def body(j, carry):
        cida = wid + (2 * j) * NW
        cidb = wid + (2 * j + 1) * NW
        cida2 = wid + (2 * j + 2) * NW

        @pl.when(cida < NCHUNK_S)
        def _():
            @pl.when(cidb < NCHUNK_S)
            def _():
                @pl.when(j > 0)
                def _():
                    adrain(1)

                issue_loads(1, cidb * CS)

            ldrain(0, cida * CS)
            build_and_add(0)

            @pl.when(cidb < NCHUNK_S)
            def _():
                ldrain(1, cidb * CS)
                build_and_add(1)
                adrain(0)

                @pl.when(cida2 < NCHUNK_S)
                def _():
                    issue_loads(0, cida2 * CS)

            @pl.when(cidb >= NCHUNK_S)
            def _():
                adrain(0)

        return carry

    lax.fori_loop(0, (CHUNKS_S + 1) // 2, body, None)
    adrain(1)
    plsc.subcore_barrier()

    def pblk(j, carry):
        b = sid + j * NS

        @pl.when(b < NBLK)
        def _():
            pltpu.sync_copy(msg_acc.at[pl.ds(b * BR, BR)], zvm)
            pltpu.sync_copy(zvm, msgp_hbm.at[pl.ds(cidx * N + b * BR, BR)])
            pltpu.sync_copy(xacc_flat.at[pl.ds(b * 640, 640)], zbf)
            pltpu.sync_copy(zbf,
                            xaccp_hbm.at[pl.ds(cidx * N * XW + b * 640, 640)])

        return carry

    lax.fori_loop(0, -(-NBLK // NS), pblk, None)


def _scatter(m, coef, dx, dy, dz, dst, zm, zf):
    cbuf = lambda: pltpu.VMEM((CS,), jnp.float32)
    sset = [pltpu.VMEM((CS,), jnp.int32), pltpu.VMEM((CS, D), jnp.float32),
            cbuf(), cbuf(), cbuf(), cbuf(),
            pltpu.VMEM((3 * CS,), jnp.float32),
            pltpu.VMEM((3 * CS,), jnp.int32)]
    f = functools.partial(
        pl.kernel,
        out_type=[jax.ShapeDtypeStruct((NC * N, D), jnp.float32),
                  jax.ShapeDtypeStruct((NC * N * XW,), jnp.float32)],
        mesh=_sc_mesh(),
        compiler_params=_SC_PARAMS,
        scratch_types=([pltpu.VMEM_SHARED((N, D), jnp.float32),
                        pltpu.VMEM_SHARED((N * XW,), jnp.float32)]
                       + sset + sset
                       + [pltpu.VMEM((BR, D), jnp.float32),
                          pltpu.VMEM((640,), jnp.float32),
                          pltpu.SemaphoreType.DMA, pltpu.SemaphoreType.DMA,
                          pltpu.SemaphoreType.DMA, pltpu.SemaphoreType.DMA]),
    )(_scatter_body)
    return f(m, coef, dx, dy, dz, dst, zm, zf)


def _node_body(h_ref, msg0_ref, msg1_ref, xp_ref, xa0_ref, xa1_ref,
               wh1h_ref, wh1m_ref, bh1_ref, wh2_ref, bh2_ref,
               hout_ref, xout_ref):
    h = h_ref[...]
    msg = msg0_ref[...] + msg1_ref[...]
    u = (jnp.dot(h, wh1h_ref[...], preferred_element_type=jnp.float32)
         + jnp.dot(msg, wh1m_ref[...], preferred_element_type=jnp.float32)
         + bh1_ref[...])
    t = u * jax.nn.sigmoid(u)
    hout_ref[...] = (h + jnp.dot(t, wh2_ref[...],
                                 preferred_element_type=jnp.float32)
                     + bh2_ref[...])
    xout_ref[...] = xp_ref[...] + xa0_ref[...] + xa1_ref[...]


def _node(h, msg0, msg1, xp, xa0, xa1, wh1h, wh1m, bh1, wh2, bh2):
    BN = 2000
    grid = N // BN
    full = lambda shape: pl.BlockSpec(shape, lambda i: (0, 0))
    return pl.pallas_call(
        _node_body,
        grid=(grid,),
        in_specs=[
            pl.BlockSpec((BN, D), lambda i: (i, 0)),
            pl.BlockSpec((BN, D), lambda i: (i, 0)),
            pl.BlockSpec((BN, D), lambda i: (i, 0)),
            pl.BlockSpec((BN, XW), lambda i: (i, 0)),
            pl.BlockSpec((BN, XW), lambda i: (i, 0)),
            pl.BlockSpec((BN, XW), lambda i: (i, 0)),
            full((D, D)), full((D, D)), full((1, D)), full((D, D)),
            full((1, D)),
        ],
        out_specs=[
            pl.BlockSpec((BN, D), lambda i: (i, 0)),
            pl.BlockSpec((BN, XW), lambda i: (i, 0)),
        ],
        out_shape=[jax.ShapeDtypeStruct((N, D), jnp.float32),
                   jax.ShapeDtypeStruct((N, XW), jnp.float32)],
    )(h, msg0, msg1, xp, xa0, xa1, wh1h, wh1m, bh1, wh2, bh2)


def kernel(h, x, edges, means, stds, W1, b1, W2, b2, Wa, ba,
           Wx1, bx1, Wx2, Wh1, bh1, Wh2, bh2):
    p, q = _pq(h, W1[:D], W1[D:2 * D])
    src_idx = edges[0]
    dst_idx = edges[1]
    ps, qd, dbx, dby, dbz, d2 = _gather(p, q, x.reshape(-1), src_idx, dst_idx)
    inv2s2 = 1.0 / (2.0 * stds * stds)
    m, coef = _edge(ps, qd, d2.reshape(E, 1),
                    means.reshape(1, DD), inv2s2.reshape(1, DD),
                    W1[2 * D:], b1.reshape(1, D), W2, b2.reshape(1, D),
                    Wa, ba.reshape(1, 1), Wx1, bx1.reshape(1, D), Wx2)
    zm = jnp.zeros((BR, D), jnp.float32)
    zf = jnp.zeros((640,), jnp.float32)
    msgp, xaccp = _scatter(m, coef.reshape(E), dbx, dby, dbz, dst_idx, zm, zf)
    msgp = msgp.reshape(NC, N, D)
    xaccp = xaccp.reshape(NC, N, XW)
    xpad = jnp.pad(x, ((0, 0), (0, XW - 3)))
    h_new, xsum = _node(h, msgp[0], msgp[1], xpad, xaccp[0], xaccp[1],
                        Wh1[:D], Wh1[D:], bh1.reshape(1, D), Wh2,
                        bh2.reshape(1, D))
    return (h_new, xsum[:, :3])


# speedup vs baseline: 4.8910x; 1.0728x over previous
import functools

import jax
import jax.numpy as jnp
from jax import lax
from jax.experimental import pallas as pl
from jax.experimental.pallas import tpu as pltpu
from jax.experimental.pallas import tpu_sc as plsc

N = 10000
D = 128
DD = 16
E = 320000
R_CUTOFF = 5.0
SPEED = 0.1

NC = 2
NS = 16
NW = NC * NS
C = 128
NCHUNK = E // C
CHUNKS_PER_W = -(-NCHUNK // NW)
CS = 32
NCHUNK_S = E // CS
CHUNKS_S = -(-NCHUNK_S // NW)
BR = 40
NBLK = N // BR
BLKS_PER_TILE = -(-NBLK // NS)

@functools.lru_cache(maxsize=None)
def _sc_mesh():
    return plsc.VectorSubcoreMesh(core_axis_name="c", subcore_axis_name="s",
                                  num_cores=NC, num_subcores=NS)


_SC_PARAMS = pltpu.CompilerParams(needs_layout_passes=False)


def _pq_body(h_ref, w1s_ref, w1d_ref, p_ref, q_ref):
    hb = h_ref[...]
    p_ref[...] = jnp.dot(hb, w1s_ref[...], preferred_element_type=jnp.float32)
    q_ref[...] = jnp.dot(hb, w1d_ref[...], preferred_element_type=jnp.float32)


def _pq(h, w1s, w1d):
    return pl.pallas_call(
        _pq_body,
        out_shape=[jax.ShapeDtypeStruct((N, D), jnp.float32),
                   jax.ShapeDtypeStruct((N, D), jnp.float32)],
    )(h, w1s, w1d)


def _gather_body(p_hbm, q_hbm, xflat_hbm, src_hbm, dst_hbm,
                 ps_hbm, qd_hbm, dbx_hbm, dby_hbm, dbz_hbm, d2_hbm,
                 xtile,
                 srcv0, dstv0, psbuf0, qdbuf0, dbx0, dby0, dbz0, d2b0,
                 srcv1, dstv1, psbuf1, qdbuf1, dbx1, dby1, dbz1, d2b1,
                 isem0, isem1, gsem0, gsem1, wsem0, wsem1):
    wid = lax.axis_index("s") * NC + lax.axis_index("c")
    pltpu.sync_copy(xflat_hbm, xtile)

    sets = ((srcv0, dstv0, psbuf0, qdbuf0, dbx0, dby0, dbz0, d2b0,
             isem0, gsem0, wsem0),
            (srcv1, dstv1, psbuf1, qdbuf1, dbx1, dby1, dbz1, d2b1,
             isem1, gsem1, wsem1))

    def issue_idx(p, off):
        (srcv, dstv, psbuf, qdbuf, dbx, dby, dbz, d2b, isem, gsem,
         wsem) = sets[p]
        pltpu.async_copy(src_hbm.at[pl.ds(off, C)], srcv, isem)
        pltpu.async_copy(dst_hbm.at[pl.ds(off, C)], dstv, isem)

    def idrain(p, off):
        (srcv, dstv, psbuf, qdbuf, dbx, dby, dbz, d2b, isem, gsem,
         wsem) = sets[p]
        pltpu.make_async_copy(src_hbm.at[pl.ds(off, C)], srcv, isem).wait()
        pltpu.make_async_copy(dst_hbm.at[pl.ds(off, C)], dstv, isem).wait()

    def wdrain(p, off):
        (srcv, dstv, psbuf, qdbuf, dbx, dby, dbz, d2b, isem, gsem,
         wsem) = sets[p]
        pltpu.make_async_copy(psbuf, ps_hbm.at[pl.ds(off, C)], wsem).wait()
        pltpu.make_async_copy(qdbuf, qd_hbm.at[pl.ds(off, C)], wsem).wait()
        pltpu.make_async_copy(dbx, dbx_hbm.at[pl.ds(off, C)], wsem).wait()
        pltpu.make_async_copy(dby, dby_hbm.at[pl.ds(off, C)], wsem).wait()
        pltpu.make_async_copy(dbz, dbz_hbm.at[pl.ds(off, C)], wsem).wait()
        pltpu.make_async_copy(d2b, d2_hbm.at[pl.ds(off, C)], wsem).wait()

    def process(p, off):
        (srcv, dstv, psbuf, qdbuf, dbx, dby, dbz, d2b, isem, gsem,
         wsem) = sets[p]
        cp1 = pltpu.async_copy(p_hbm.at[srcv], psbuf, gsem)
        cp2 = pltpu.async_copy(q_hbm.at[dstv], qdbuf, gsem)
        for k in range(C // 16):
            sl = pl.ds(k * 16, 16)
            s16 = srcv[sl] * 3
            d16 = dstv[sl] * 3
            dx = (plsc.load_gather(xtile, [d16])
                  - plsc.load_gather(xtile, [s16]))
            dy = (plsc.load_gather(xtile, [d16 + 1])
                  - plsc.load_gather(xtile, [s16 + 1]))
            dz = (plsc.load_gather(xtile, [d16 + 2])
                  - plsc.load_gather(xtile, [s16 + 2]))
            dbx[sl] = dx
            dby[sl] = dy
            dbz[sl] = dz
            d2b[sl] = dx * dx + dy * dy + dz * dz
        cp1.wait()
        cp2.wait()
        pltpu.async_copy(psbuf, ps_hbm.at[pl.ds(off, C)], wsem)
        pltpu.async_copy(qdbuf, qd_hbm.at[pl.ds(off, C)], wsem)
        pltpu.async_copy(dbx, dbx_hbm.at[pl.ds(off, C)], wsem)
        pltpu.async_copy(dby, dby_hbm.at[pl.ds(off, C)], wsem)
        pltpu.async_copy(dbz, dbz_hbm.at[pl.ds(off, C)], wsem)
        pltpu.async_copy(d2b, d2_hbm.at[pl.ds(off, C)], wsem)

    issue_idx(0, wid * C)

    def body(j, carry):
        cida = wid + (2 * j) * NW
        cidb = wid + (2 * j + 1) * NW
        cida2 = wid + (2 * j + 2) * NW

        @pl.when(cida < NCHUNK)
        def _():
            offa = cida * C

            @pl.when(cidb < NCHUNK)
            def _():
                issue_idx(1, cidb * C)

            @pl.when(j > 0)
            def _():
                wdrain(0, offa)

            idrain(0, offa)
            process(0, offa)

            @pl.when(cidb < NCHUNK)
            def _():
                offb = cidb * C

                @pl.when(j > 0)
                def _():
                    wdrain(1, offb)

                idrain(1, offb)

                @pl.when(cida2 < NCHUNK)
                def _():
                    issue_idx(0, cida2 * C)

                process(1, offb)

        return carry

    lax.fori_loop(0, (CHUNKS_PER_W + 1) // 2, body, None)
    wdrain(0, wid * C)
    wdrain(1, wid * C)


def _gather(p, q, xflat, src, dst):
    gset = [pltpu.VMEM((C,), jnp.int32), pltpu.VMEM((C,), jnp.int32),
            pltpu.VMEM((C, D), jnp.float32), pltpu.VMEM((C, D), jnp.float32),
            pltpu.VMEM((C,), jnp.float32), pltpu.VMEM((C,), jnp.float32),
            pltpu.VMEM((C,), jnp.float32), pltpu.VMEM((C,), jnp.float32)]
    f = functools.partial(
        pl.kernel,
        out_type=[jax.ShapeDtypeStruct((E, D), jnp.float32),
                  jax.ShapeDtypeStruct((E, D), jnp.float32),
                  jax.ShapeDtypeStruct((E,), jnp.float32),
                  jax.ShapeDtypeStruct((E,), jnp.float32),
                  jax.ShapeDtypeStruct((E,), jnp.float32),
                  jax.ShapeDtypeStruct((E,), jnp.float32)],
        mesh=_sc_mesh(),
        compiler_params=_SC_PARAMS,
        scratch_types=([pltpu.VMEM((N * 3,), jnp.float32)] + gset + gset
                      + [pltpu.SemaphoreType.DMA] * 6),
    )(_gather_body)
    return f(p, q, xflat, src, dst)


BE = 4000


def _edge_body(ps_ref, qd_ref, d2_ref, means_ref, inv2s2_ref,
               w1r_ref, b1_ref, w2_ref, b2_ref, wa_ref, ba_ref,
               wx1_ref, bx1_ref, wx2_ref,
               m_ref, coef_ref):
    d2 = d2_ref[...]
    dist = jnp.sqrt(d2 + 1e-12)
    valid = (dist < R_CUTOFF).astype(jnp.float32)
    delta = dist - means_ref[...]
    rbf = jnp.exp(-delta * delta * inv2s2_ref[...])
    u = (ps_ref[...] + qd_ref[...] + b1_ref[...]
         + jnp.dot(rbf, w1r_ref[...], preferred_element_type=jnp.float32))
    m1 = u * jax.nn.sigmoid(u)
    v = jnp.dot(m1, w2_ref[...], preferred_element_type=jnp.float32) + b2_ref[...]
    m2 = v * jax.nn.sigmoid(v)
    att = jax.nn.sigmoid(
        jnp.dot(m2, wa_ref[...], preferred_element_type=jnp.float32) + ba_ref[...])
    m_att = m2 * att
    m_ref[...] = m_att * valid
    g = jnp.dot(m_att, wx1_ref[...], preferred_element_type=jnp.float32) + bx1_ref[...]
    g = g * jax.nn.sigmoid(g)
    mag = jnp.tanh(jnp.dot(g, wx2_ref[...], preferred_element_type=jnp.float32))
    coef_ref[...] = SPEED * valid * mag / dist


def _edge(ps, qd, d2, means, inv2s2, w1r, b1, w2, b2, wa, ba, wx1, bx1, wx2):
    grid = E // BE
    full = lambda shape: pl.BlockSpec(shape, lambda i: (0, 0))
    return pl.pallas_call(
        _edge_body,
        grid=(grid,),
        in_specs=[
            pl.BlockSpec((BE, D), lambda i: (i, 0)),
            pl.BlockSpec((BE, D), lambda i: (i, 0)),
            pl.BlockSpec((BE, 1), lambda i: (i, 0)),
            full((1, DD)), full((1, DD)),
            full((DD, D)), full((1, D)), full((D, D)), full((1, D)),
            full((D, 1)), full((1, 1)),
            full((D, D)), full((1, D)), full((D, 1)),
        ],
        out_specs=[
            pl.BlockSpec((BE, D), lambda i: (i, 0)),
            pl.BlockSpec((BE, 1), lambda i: (i, 0)),
        ],
        out_shape=[jax.ShapeDtypeStruct((E, D), jnp.float32),
                   jax.ShapeDtypeStruct((E, 1), jnp.float32)],
    )(ps, qd, d2, means, inv2s2, w1r, b1, w2, b2, wa, ba, wx1, bx1, wx2)


XW = 16
NBX = N * XW // 640


def _scatter_body(m_hbm, coef_hbm, dx_hbm, dy_hbm, dz_hbm, dst_hbm,
                  zm_hbm, zf_hbm,
                  msgp_hbm, xaccp_hbm,
                  msg_acc, xacc_flat,
                  dstv0, mv0, cv0, dfx0, dfy0, dfz0, dsp0, dsi0,
                  dstv1, mv1, cv1, dfx1, dfy1, dfz1, dsp1, dsi1,
                  zvm, zbf,
                  lsem0, lsem1, asem0, asem1):
    cidx = lax.axis_index("c")
    sid = lax.axis_index("s")
    wid = sid * NC + cidx

    sets = ((dstv0, mv0, cv0, dfx0, dfy0, dfz0, dsp0, dsi0, lsem0, asem0),
            (dstv1, mv1, cv1, dfx1, dfy1, dfz1, dsp1, dsi1, lsem1, asem1))

    pltpu.sync_copy(zm_hbm, zvm)
    pltpu.sync_copy(zf_hbm, zbf)

    def zblk(j, carry):
        b = sid + j * NS

        @pl.when(b < NBLK)
        def _():
            pltpu.sync_copy(zvm, msg_acc.at[pl.ds(b * BR, BR)])
            pltpu.sync_copy(zbf, xacc_flat.at[pl.ds(b * 640, 640)])

        return carry

    lax.fori_loop(0, -(-NBLK // NS), zblk, None)
    plsc.subcore_barrier()

    iota16 = lax.iota(jnp.int32, 16)

    def issue_loads(p, off):
        (dstv, mv, cv, dfx, dfy, dfz, dsp, dsi, lsem, asem) = sets[p]
        pltpu.async_copy(dst_hbm.at[pl.ds(off, CS)], dstv, lsem)
        pltpu.async_copy(m_hbm.at[pl.ds(off, CS)], mv, lsem)
        pltpu.async_copy(coef_hbm.at[pl.ds(off, CS)], cv, lsem)
        pltpu.async_copy(dx_hbm.at[pl.ds(off, CS)], dfx, lsem)
        pltpu.async_copy(dy_hbm.at[pl.ds(off, CS)], dfy, lsem)
        pltpu.async_copy(dz_hbm.at[pl.ds(off, CS)], dfz, lsem)

    def ldrain(p, off):
        (dstv, mv, cv, dfx, dfy, dfz, dsp, dsi, lsem, asem) = sets[p]
        pltpu.make_async_copy(dst_hbm.at[pl.ds(off, CS)], dstv, lsem).wait()
        pltpu.make_async_copy(m_hbm.at[pl.ds(off, CS)], mv, lsem).wait()
        pltpu.make_async_copy(coef_hbm.at[pl.ds(off, CS)], cv, lsem).wait()
        pltpu.make_async_copy(dx_hbm.at[pl.ds(off, CS)], dfx, lsem).wait()
        pltpu.make_async_copy(dy_hbm.at[pl.ds(off, CS)], dfy, lsem).wait()
        pltpu.make_async_copy(dz_hbm.at[pl.ds(off, CS)], dfz, lsem).wait()

    def build_and_add(p):
        (dstv, mv, cv, dfx, dfy, dfz, dsp, dsi, lsem, asem) = sets[p]
        for k in range(CS // 16):
            sl = pl.ds(k * 16, 16)
            d16 = dstv[sl]
            c16 = cv[sl]
            rows3 = (k * 16 + iota16) * 3
            for c, buf in ((0, dfx), (1, dfy), (2, dfz)):
                plsc.store_scatter(dsi, [rows3 + c], d16 * XW + c)
                plsc.store_scatter(dsp, [rows3 + c], c16 * buf[sl])
        pltpu.async_copy(mv, msg_acc.at[dstv], asem, add=True)
        pltpu.async_copy(dsp, xacc_flat.at[dsi], asem, add=True)

    def adrain(p):
        (dstv, mv, cv, dfx, dfy, dfz, dsp, dsi, lsem, asem) = sets[p]
        pltpu.make_async_copy(mv, msg_acc.at[dstv], asem).wait()
        pltpu.make_async_copy(dsp, xacc_flat.at[dsi], asem).wait()

    issue_loads(0, wid * CS)

    def body(j, carry):
        cida = wid + (2 * j) * NW
        cidb = wid + (2 * j + 1) * NW
        cida2 = wid + (2 * j + 2) * NW

        @pl.when(cida < NCHUNK_S)
        def _():
            @pl.when(cidb < NCHUNK_S)
            def _():
                @pl.when(j > 0)
                def _():
                    adrain(1)

                issue_loads(1, cidb * CS)

            ldrain(0, cida * CS)
            build_and_add(0)

            @pl.when(cidb < NCHUNK_S)
            def _():
                ldrain(1, cidb * CS)
                build_and_add(1)
                adrain(0)

                @pl.when(cida2 < NCHUNK_S)
                def _():
                    issue_loads(0, cida2 * CS)

            @pl.when(cidb >= NCHUNK_S)
            def _():
                adrain(0)

        return carry

    lax.fori_loop(0, (CHUNKS_S + 1) // 2, body, None)
    adrain(1)
    plsc.subcore_barrier()

    def pblk(j, carry):
        b = sid + j * NS

        @pl.when(b < NBLK)
        def _():
            pltpu.sync_copy(msg_acc.at[pl.ds(b * BR, BR)], zvm)
            pltpu.sync_copy(zvm, msgp_hbm.at[pl.ds(cidx * N + b * BR, BR)])
            pltpu.sync_copy(xacc_flat.at[pl.ds(b * 640, 640)], zbf)
            pltpu.sync_copy(zbf,
                            xaccp_hbm.at[pl.ds(cidx * N * XW + b * 640, 640)])

        return carry

    lax.fori_loop(0, -(-NBLK // NS), pblk, None)


def _scatter(m, coef, dx, dy, dz, dst, zm, zf):
    cbuf = lambda: pltpu.VMEM((CS,), jnp.float32)
    sset = [pltpu.VMEM((CS,), jnp.int32), pltpu.VMEM((CS, D), jnp.float32),
            cbuf(), cbuf(), cbuf(), cbuf(),
            pltpu.VMEM((3 * CS,), jnp.float32),
            pltpu.VMEM((3 * CS,), jnp.int32)]
    f = functools.partial(
        pl.kernel,
        out_type=[jax.ShapeDtypeStruct((NC * N, D), jnp.float32),
                  jax.ShapeDtypeStruct((NC * N * XW,), jnp.float32)],
        mesh=_sc_mesh(),
        compiler_params=_SC_PARAMS,
        scratch_types=([pltpu.VMEM_SHARED((N, D), jnp.float32),
                        pltpu.VMEM_SHARED((N * XW,), jnp.float32)]
                       + sset + sset
                       + [pltpu.VMEM((BR, D), jnp.float32),
                          pltpu.VMEM((640,), jnp.float32),
                          pltpu.SemaphoreType.DMA, pltpu.SemaphoreType.DMA,
                          pltpu.SemaphoreType.DMA, pltpu.SemaphoreType.DMA]),
    )(_scatter_body)
    return f(m, coef, dx, dy, dz, dst, zm, zf)


def _node_body(h_ref, msg0_ref, msg1_ref, xp_ref, xa0_ref, xa1_ref,
               wh1h_ref, wh1m_ref, bh1_ref, wh2_ref, bh2_ref,
               hout_ref, xout_ref):
    h = h_ref[...]
    msg = msg0_ref[...] + msg1_ref[...]
    u = (jnp.dot(h, wh1h_ref[...], preferred_element_type=jnp.float32)
         + jnp.dot(msg, wh1m_ref[...], preferred_element_type=jnp.float32)
         + bh1_ref[...])
    t = u * jax.nn.sigmoid(u)
    hout_ref[...] = (h + jnp.dot(t, wh2_ref[...],
                                 preferred_element_type=jnp.float32)
                     + bh2_ref[...])
    xout_ref[...] = xp_ref[...] + xa0_ref[...] + xa1_ref[...]


def _node(h, msg0, msg1, xp, xa0, xa1, wh1h, wh1m, bh1, wh2, bh2):
    BN = 2000
    grid = N // BN
    full = lambda shape: pl.BlockSpec(shape, lambda i: (0, 0))
    return pl.pallas_call(
        _node_body,
        grid=(grid,),
        in_specs=[
            pl.BlockSpec((BN, D), lambda i: (i, 0)),
            pl.BlockSpec((BN, D), lambda i: (i, 0)),
            pl.BlockSpec((BN, D), lambda i: (i, 0)),
            pl.BlockSpec((BN, XW), lambda i: (i, 0)),
            pl.BlockSpec((BN, XW), lambda i: (i, 0)),
            pl.BlockSpec((BN, XW), lambda i: (i, 0)),
            full((D, D)), full((D, D)), full((1, D)), full((D, D)),
            full((1, D)),
        ],
        out_specs=[
            pl.BlockSpec((BN, D), lambda i: (i, 0)),
            pl.BlockSpec((BN, XW), lambda i: (i, 0)),
        ],
        out_shape=[jax.ShapeDtypeStruct((N, D), jnp.float32),
                   jax.ShapeDtypeStruct((N, XW), jnp.float32)],
    )(h, msg0, msg1, xp, xa0, xa1, wh1h, wh1m, bh1, wh2, bh2)


def kernel(h, x, edges, means, stds, W1, b1, W2, b2, Wa, ba,
           Wx1, bx1, Wx2, Wh1, bh1, Wh2, bh2):
    p, q = _pq(h, W1[:D], W1[D:2 * D])
    src_idx = edges[0]
    dst_idx = edges[1]
    ps, qd, dbx, dby, dbz, d2 = _gather(p, q, x.reshape(-1), src_idx, dst_idx)
    inv2s2 = 1.0 / (2.0 * stds * stds)
    m, coef = _edge(ps, qd, d2.reshape(E, 1),
                    means.reshape(1, DD), inv2s2.reshape(1, DD),
                    W1[2 * D:], b1.reshape(1, D), W2, b2.reshape(1, D),
                    Wa, ba.reshape(1, 1), Wx1, bx1.reshape(1, D), Wx2)
    zm = jnp.zeros((BR, D), jnp.float32)
    zf = jnp.zeros((640,), jnp.float32)
    msgp, xaccp = _scatter(m, coef.reshape(E), dbx, dby, dbz, dst_idx, zm, zf)
    msgp = msgp.reshape(NC, N, D)
    xaccp = xaccp.reshape(NC, N, XW)
    xpad = jnp.pad(x, ((0, 0), (0, XW - 3)))
    h_new, xsum = _node(h, msgp[0], msgp[1], xpad, xaccp[0], xaccp[1],
                        Wh1[:D], Wh1[D:], bh1.reshape(1, D), Wh2,
                        bh2.reshape(1, D))
    return (h_new, xsum[:, :3])


# speedup vs baseline: 5.4202x; 1.1082x over previous
import functools

import jax
import jax.numpy as jnp
from jax import lax
from jax.experimental import pallas as pl
from jax.experimental.pallas import tpu as pltpu
from jax.experimental.pallas import tpu_sc as plsc

N = 10000
D = 128
DD = 16
E = 320000
R_CUTOFF = 5.0
SPEED = 0.1

NC = 2
NS = 16
NW = NC * NS
C = 128
NCHUNK = E // C
CHUNKS_PER_W = -(-NCHUNK // NW)
CS = 32
NCHUNK_S = E // CS
CHUNKS_S = -(-NCHUNK_S // NW)
BR = 40
NBLK = N // BR
BLKS_PER_TILE = -(-NBLK // NS)

@functools.lru_cache(maxsize=None)
def _sc_mesh():
    return plsc.VectorSubcoreMesh(core_axis_name="c", subcore_axis_name="s",
                                  num_cores=NC, num_subcores=NS)


_SC_PARAMS = pltpu.CompilerParams(needs_layout_passes=False)


def _pq_body(h_ref, w1s_ref, w1d_ref, p_ref, q_ref):
    hb = h_ref[...]
    p_ref[...] = jnp.dot(hb, w1s_ref[...], preferred_element_type=jnp.float32)
    q_ref[...] = jnp.dot(hb, w1d_ref[...], preferred_element_type=jnp.float32)


def _pq(h, w1s, w1d):
    return pl.pallas_call(
        _pq_body,
        out_shape=[jax.ShapeDtypeStruct((N, D), jnp.float32),
                   jax.ShapeDtypeStruct((N, D), jnp.float32)],
    )(h, w1s, w1d)


def _gather_body(nchunk, chunks_pw, p_hbm, q_hbm, xflat_hbm, src_hbm, dst_hbm,
                 ps_hbm, qd_hbm, dbx_hbm, dby_hbm, dbz_hbm, d2_hbm,
                 xtile,
                 srcv0, dstv0, psbuf0, qdbuf0, dbx0, dby0, dbz0, d2b0,
                 srcv1, dstv1, psbuf1, qdbuf1, dbx1, dby1, dbz1, d2b1,
                 isem0, isem1, gsem0, gsem1, wsem0, wsem1):
    wid = lax.axis_index("s") * NC + lax.axis_index("c")
    pltpu.sync_copy(xflat_hbm, xtile)

    sets = ((srcv0, dstv0, psbuf0, qdbuf0, dbx0, dby0, dbz0, d2b0,
             isem0, gsem0, wsem0),
            (srcv1, dstv1, psbuf1, qdbuf1, dbx1, dby1, dbz1, d2b1,
             isem1, gsem1, wsem1))

    def issue_idx(p, off):
        (srcv, dstv, psbuf, qdbuf, dbx, dby, dbz, d2b, isem, gsem,
         wsem) = sets[p]
        pltpu.async_copy(src_hbm.at[pl.ds(off, C)], srcv, isem)
        pltpu.async_copy(dst_hbm.at[pl.ds(off, C)], dstv, isem)

    def idrain(p, off):
        (srcv, dstv, psbuf, qdbuf, dbx, dby, dbz, d2b, isem, gsem,
         wsem) = sets[p]
        pltpu.make_async_copy(src_hbm.at[pl.ds(off, C)], srcv, isem).wait()
        pltpu.make_async_copy(dst_hbm.at[pl.ds(off, C)], dstv, isem).wait()

    def wdrain(p, off):
        (srcv, dstv, psbuf, qdbuf, dbx, dby, dbz, d2b, isem, gsem,
         wsem) = sets[p]
        pltpu.make_async_copy(psbuf, ps_hbm.at[pl.ds(off, C)], wsem).wait()
        pltpu.make_async_copy(qdbuf, qd_hbm.at[pl.ds(off, C)], wsem).wait()
        pltpu.make_async_copy(dbx, dbx_hbm.at[pl.ds(off, C)], wsem).wait()
        pltpu.make_async_copy(dby, dby_hbm.at[pl.ds(off, C)], wsem).wait()
        pltpu.make_async_copy(dbz, dbz_hbm.at[pl.ds(off, C)], wsem).wait()
        pltpu.make_async_copy(d2b, d2_hbm.at[pl.ds(off, C)], wsem).wait()

    def process(p, off):
        (srcv, dstv, psbuf, qdbuf, dbx, dby, dbz, d2b, isem, gsem,
         wsem) = sets[p]
        cp1 = pltpu.async_copy(p_hbm.at[srcv], psbuf, gsem)
        cp2 = pltpu.async_copy(q_hbm.at[dstv], qdbuf, gsem)
        for k in range(C // 16):
            sl = pl.ds(k * 16, 16)
            s16 = srcv[sl] * 3
            d16 = dstv[sl] * 3
            dx = (plsc.load_gather(xtile, [d16])
                  - plsc.load_gather(xtile, [s16]))
            dy = (plsc.load_gather(xtile, [d16 + 1])
                  - plsc.load_gather(xtile, [s16 + 1]))
            dz = (plsc.load_gather(xtile, [d16 + 2])
                  - plsc.load_gather(xtile, [s16 + 2]))
            dbx[sl] = dx
            dby[sl] = dy
            dbz[sl] = dz
            d2b[sl] = dx * dx + dy * dy + dz * dz
        cp1.wait()
        cp2.wait()
        pltpu.async_copy(psbuf, ps_hbm.at[pl.ds(off, C)], wsem)
        pltpu.async_copy(qdbuf, qd_hbm.at[pl.ds(off, C)], wsem)
        pltpu.async_copy(dbx, dbx_hbm.at[pl.ds(off, C)], wsem)
        pltpu.async_copy(dby, dby_hbm.at[pl.ds(off, C)], wsem)
        pltpu.async_copy(dbz, dbz_hbm.at[pl.ds(off, C)], wsem)
        pltpu.async_copy(d2b, d2_hbm.at[pl.ds(off, C)], wsem)

    issue_idx(0, wid * C)

    def body(j, carry):
        cida = wid + (2 * j) * NW
        cidb = wid + (2 * j + 1) * NW
        cida2 = wid + (2 * j + 2) * NW

        @pl.when(cida < nchunk)
        def _():
            offa = cida * C

            @pl.when(cidb < nchunk)
            def _():
                issue_idx(1, cidb * C)

            @pl.when(j > 0)
            def _():
                wdrain(0, offa)

            idrain(0, offa)
            process(0, offa)

            @pl.when(cidb < nchunk)
            def _():
                offb = cidb * C

                @pl.when(j > 0)
                def _():
                    wdrain(1, offb)

                idrain(1, offb)

                @pl.when(cida2 < nchunk)
                def _():
                    issue_idx(0, cida2 * C)

                process(1, offb)

        return carry

    lax.fori_loop(0, (chunks_pw + 1) // 2, body, None)
    wdrain(0, wid * C)
    wdrain(1, wid * C)


def _gather(p, q, xflat, src, dst):
    ne = src.shape[0]
    nchunk = ne // C
    chunks_pw = -(-nchunk // NW)
    gset = [pltpu.VMEM((C,), jnp.int32), pltpu.VMEM((C,), jnp.int32),
            pltpu.VMEM((C, D), jnp.float32), pltpu.VMEM((C, D), jnp.float32),
            pltpu.VMEM((C,), jnp.float32), pltpu.VMEM((C,), jnp.float32),
            pltpu.VMEM((C,), jnp.float32), pltpu.VMEM((C,), jnp.float32)]
    f = functools.partial(
        pl.kernel,
        out_type=[jax.ShapeDtypeStruct((ne, D), jnp.float32),
                  jax.ShapeDtypeStruct((ne, D), jnp.float32),
                  jax.ShapeDtypeStruct((ne,), jnp.float32),
                  jax.ShapeDtypeStruct((ne,), jnp.float32),
                  jax.ShapeDtypeStruct((ne,), jnp.float32),
                  jax.ShapeDtypeStruct((ne,), jnp.float32)],
        mesh=_sc_mesh(),
        compiler_params=_SC_PARAMS,
        scratch_types=([pltpu.VMEM((N * 3,), jnp.float32)] + gset + gset
                      + [pltpu.SemaphoreType.DMA] * 6),
    )(functools.partial(_gather_body, nchunk, chunks_pw))
    return f(p, q, xflat, src, dst)


BE = 4000


def _edge_body(ps_ref, qd_ref, d2_ref, means_ref, inv2s2_ref,
               w1r_ref, b1_ref, w2_ref, b2_ref, wa_ref, ba_ref,
               wx1_ref, bx1_ref, wx2_ref,
               m_ref, coef_ref):
    d2 = d2_ref[...]
    dist = jnp.sqrt(d2 + 1e-12)
    valid = (dist < R_CUTOFF).astype(jnp.float32)
    delta = dist - means_ref[...]
    rbf = jnp.exp(-delta * delta * inv2s2_ref[...])
    u = (ps_ref[...] + qd_ref[...] + b1_ref[...]
         + jnp.dot(rbf, w1r_ref[...], preferred_element_type=jnp.float32))
    m1 = u * jax.nn.sigmoid(u)
    v = jnp.dot(m1, w2_ref[...], preferred_element_type=jnp.float32) + b2_ref[...]
    m2 = v * jax.nn.sigmoid(v)
    att = jax.nn.sigmoid(
        jnp.dot(m2, wa_ref[...], preferred_element_type=jnp.float32) + ba_ref[...])
    m_att = m2 * att
    m_ref[...] = m_att * valid
    g = jnp.dot(m_att, wx1_ref[...], preferred_element_type=jnp.float32) + bx1_ref[...]
    g = g * jax.nn.sigmoid(g)
    mag = jnp.tanh(jnp.dot(g, wx2_ref[...], preferred_element_type=jnp.float32))
    coef_ref[...] = SPEED * valid * mag / dist


def _edge(ps, qd, d2, means, inv2s2, w1r, b1, w2, b2, wa, ba, wx1, bx1, wx2):
    ne = ps.shape[0]
    grid = ne // BE
    full = lambda shape: pl.BlockSpec(shape, lambda i: (0, 0))
    return pl.pallas_call(
        _edge_body,
        grid=(grid,),
        in_specs=[
            pl.BlockSpec((BE, D), lambda i: (i, 0)),
            pl.BlockSpec((BE, D), lambda i: (i, 0)),
            pl.BlockSpec((BE, 1), lambda i: (i, 0)),
            full((1, DD)), full((1, DD)),
            full((DD, D)), full((1, D)), full((D, D)), full((1, D)),
            full((D, 1)), full((1, 1)),
            full((D, D)), full((1, D)), full((D, 1)),
        ],
        out_specs=[
            pl.BlockSpec((BE, D), lambda i: (i, 0)),
            pl.BlockSpec((BE, 1), lambda i: (i, 0)),
        ],
        out_shape=[jax.ShapeDtypeStruct((ne, D), jnp.float32),
                   jax.ShapeDtypeStruct((ne, 1), jnp.float32)],
    )(ps, qd, d2, means, inv2s2, w1r, b1, w2, b2, wa, ba, wx1, bx1, wx2)


XW = 16
NBX = N * XW // 640


def _scatter_body(nchunk_s, chunks_s, m_hbm, coef_hbm, dx_hbm, dy_hbm, dz_hbm, dst_hbm,
                  zm_hbm, zf_hbm,
                  msgp_hbm, xaccp_hbm,
                  msg_acc, xacc_flat,
                  dstv0, mv0, cv0, dfx0, dfy0, dfz0, dsp0, dsi0,
                  dstv1, mv1, cv1, dfx1, dfy1, dfz1, dsp1, dsi1,
                  zvm, zbf,
                  lsem0, lsem1, asem0, asem1):
    cidx = lax.axis_index("c")
    sid = lax.axis_index("s")
    wid = sid * NC + cidx

    sets = ((dstv0, mv0, cv0, dfx0, dfy0, dfz0, dsp0, dsi0, lsem0, asem0),
            (dstv1, mv1, cv1, dfx1, dfy1, dfz1, dsp1, dsi1, lsem1, asem1))

    pltpu.sync_copy(zm_hbm, zvm)
    pltpu.sync_copy(zf_hbm, zbf)

    def zblk(j, carry):
        b = sid + j * NS

        @pl.when(b < NBLK)
        def _():
            pltpu.sync_copy(zvm, msg_acc.at[pl.ds(b * BR, BR)])
            pltpu.sync_copy(zbf, xacc_flat.at[pl.ds(b * 640, 640)])

        return carry

    lax.fori_loop(0, -(-NBLK // NS), zblk, None)
    plsc.subcore_barrier()

    iota16 = lax.iota(jnp.int32, 16)

    def issue_loads(p, off):
        (dstv, mv, cv, dfx, dfy, dfz, dsp, dsi, lsem, asem) = sets[p]
        pltpu.async_copy(dst_hbm.at[pl.ds(off, CS)], dstv, lsem)
        pltpu.async_copy(m_hbm.at[pl.ds(off, CS)], mv, lsem)
        pltpu.async_copy(coef_hbm.at[pl.ds(off, CS)], cv, lsem)
        pltpu.async_copy(dx_hbm.at[pl.ds(off, CS)], dfx, lsem)
        pltpu.async_copy(dy_hbm.at[pl.ds(off, CS)], dfy, lsem)
        pltpu.async_copy(dz_hbm.at[pl.ds(off, CS)], dfz, lsem)

    def ldrain(p, off):
        (dstv, mv, cv, dfx, dfy, dfz, dsp, dsi, lsem, asem) = sets[p]
        pltpu.make_async_copy(dst_hbm.at[pl.ds(off, CS)], dstv, lsem).wait()
        pltpu.make_async_copy(m_hbm.at[pl.ds(off, CS)], mv, lsem).wait()
        pltpu.make_async_copy(coef_hbm.at[pl.ds(off, CS)], cv, lsem).wait()
        pltpu.make_async_copy(dx_hbm.at[pl.ds(off, CS)], dfx, lsem).wait()
        pltpu.make_async_copy(dy_hbm.at[pl.ds(off, CS)], dfy, lsem).wait()
        pltpu.make_async_copy(dz_hbm.at[pl.ds(off, CS)], dfz, lsem).wait()

    def build_and_add(p):
        (dstv, mv, cv, dfx, dfy, dfz, dsp, dsi, lsem, asem) = sets[p]
        for k in range(CS // 16):
            sl = pl.ds(k * 16, 16)
            d16 = dstv[sl]
            c16 = cv[sl]
            rows3 = (k * 16 + iota16) * 3
            for c, buf in ((0, dfx), (1, dfy), (2, dfz)):
                plsc.store_scatter(dsi, [rows3 + c], d16 * XW + c)
                plsc.store_scatter(dsp, [rows3 + c], c16 * buf[sl])
        pltpu.async_copy(mv, msg_acc.at[dstv], asem, add=True)
        pltpu.async_copy(dsp, xacc_flat.at[dsi], asem, add=True)

    def adrain(p):
        (dstv, mv, cv, dfx, dfy, dfz, dsp, dsi, lsem, asem) = sets[p]
        pltpu.make_async_copy(mv, msg_acc.at[dstv], asem).wait()
        pltpu.make_async_copy(dsp, xacc_flat.at[dsi], asem).wait()

    issue_loads(0, wid * CS)

    def body(j, carry):
        cida = wid + (2 * j) * NW
        cidb = wid + (2 * j + 1) * NW
        cida2 = wid + (2 * j + 2) * NW

        @pl.when(cida < nchunk_s)
        def _():
            @pl.when(cidb < nchunk_s)
            def _():
                @pl.when(j > 0)
                def _():
                    adrain(1)

                issue_loads(1, cidb * CS)

            ldrain(0, cida * CS)
            build_and_add(0)

            @pl.when(cidb < nchunk_s)
            def _():
                ldrain(1, cidb * CS)
                build_and_add(1)
                adrain(0)

                @pl.when(cida2 < nchunk_s)
                def _():
                    issue_loads(0, cida2 * CS)

            @pl.when(cidb >= nchunk_s)
            def _():
                adrain(0)

        return carry

    lax.fori_loop(0, (chunks_s + 1) // 2, body, None)
    adrain(1)
    plsc.subcore_barrier()

    def pblk(j, carry):
        b = sid + j * NS

        @pl.when(b < NBLK)
        def _():
            pltpu.sync_copy(msg_acc.at[pl.ds(b * BR, BR)], zvm)
            pltpu.sync_copy(zvm, msgp_hbm.at[pl.ds(cidx * N + b * BR, BR)])
            pltpu.sync_copy(xacc_flat.at[pl.ds(b * 640, 640)], zbf)
            pltpu.sync_copy(zbf,
                            xaccp_hbm.at[pl.ds(cidx * N * XW + b * 640, 640)])

        return carry

    lax.fori_loop(0, -(-NBLK // NS), pblk, None)


def _scatter(m, coef, dx, dy, dz, dst, zm, zf):
    ne = m.shape[0]
    nchunk_s = ne // CS
    chunks_s = -(-nchunk_s // NW)
    cbuf = lambda: pltpu.VMEM((CS,), jnp.float32)
    sset = [pltpu.VMEM((CS,), jnp.int32), pltpu.VMEM((CS, D), jnp.float32),
            cbuf(), cbuf(), cbuf(), cbuf(),
            pltpu.VMEM((3 * CS,), jnp.float32),
            pltpu.VMEM((3 * CS,), jnp.int32)]
    f = functools.partial(
        pl.kernel,
        out_type=[jax.ShapeDtypeStruct((NC * N, D), jnp.float32),
                  jax.ShapeDtypeStruct((NC * N * XW,), jnp.float32)],
        mesh=_sc_mesh(),
        compiler_params=_SC_PARAMS,
        scratch_types=([pltpu.VMEM_SHARED((N, D), jnp.float32),
                        pltpu.VMEM_SHARED((N * XW,), jnp.float32)]
                       + sset + sset
                       + [pltpu.VMEM((BR, D), jnp.float32),
                          pltpu.VMEM((640,), jnp.float32),
                          pltpu.SemaphoreType.DMA, pltpu.SemaphoreType.DMA,
                          pltpu.SemaphoreType.DMA, pltpu.SemaphoreType.DMA]),
    )(functools.partial(_scatter_body, nchunk_s, chunks_s))
    return f(m, coef, dx, dy, dz, dst, zm, zf)


def _node_body(h_ref, msg0_ref, msg1_ref, msg2_ref, msg3_ref,
               xp_ref, xa0_ref, xa1_ref, xa2_ref, xa3_ref,
               wh1h_ref, wh1m_ref, bh1_ref, wh2_ref, bh2_ref,
               hout_ref, xout_ref):
    h = h_ref[...]
    msg = ((msg0_ref[...] + msg1_ref[...])
           + (msg2_ref[...] + msg3_ref[...]))
    u = (jnp.dot(h, wh1h_ref[...], preferred_element_type=jnp.float32)
         + jnp.dot(msg, wh1m_ref[...], preferred_element_type=jnp.float32)
         + bh1_ref[...])
    t = u * jax.nn.sigmoid(u)
    hout_ref[...] = (h + jnp.dot(t, wh2_ref[...],
                                 preferred_element_type=jnp.float32)
                     + bh2_ref[...])
    xout_ref[...] = (xp_ref[...] + (xa0_ref[...] + xa1_ref[...])
                     + (xa2_ref[...] + xa3_ref[...]))


def _node(h, msgs, xp, xas, wh1h, wh1m, bh1, wh2, bh2):
    BN = 2000
    grid = N // BN
    full = lambda shape: pl.BlockSpec(shape, lambda i: (0, 0))
    return pl.pallas_call(
        _node_body,
        grid=(grid,),
        in_specs=([pl.BlockSpec((BN, D), lambda i: (i, 0))] * 5
                  + [pl.BlockSpec((BN, XW), lambda i: (i, 0))] * 5
                  + [full((D, D)), full((D, D)), full((1, D)), full((D, D)),
                     full((1, D))]),
        out_specs=[
            pl.BlockSpec((BN, D), lambda i: (i, 0)),
            pl.BlockSpec((BN, XW), lambda i: (i, 0)),
        ],
        out_shape=[jax.ShapeDtypeStruct((N, D), jnp.float32),
                   jax.ShapeDtypeStruct((N, XW), jnp.float32)],
    )(h, *msgs, xp, *xas, wh1h, wh1m, bh1, wh2, bh2)


def kernel(h, x, edges, means, stds, W1, b1, W2, b2, Wa, ba,
           Wx1, bx1, Wx2, Wh1, bh1, Wh2, bh2):
    p, q = _pq(h, W1[:D], W1[D:2 * D])
    xflat = x.reshape(-1)
    inv2s2 = 1.0 / (2.0 * stds * stds)
    zm = jnp.zeros((BR, D), jnp.float32)
    zf = jnp.zeros((640,), jnp.float32)
    he = E // 2
    msgs, xas = [], []
    for lo in (0, he):
        src_idx = lax.slice_in_dim(edges[0], lo, lo + he)
        dst_idx = lax.slice_in_dim(edges[1], lo, lo + he)
        ps, qd, dbx, dby, dbz, d2 = _gather(p, q, xflat, src_idx, dst_idx)
        m, coef = _edge(ps, qd, d2.reshape(he, 1),
                        means.reshape(1, DD), inv2s2.reshape(1, DD),
                        W1[2 * D:], b1.reshape(1, D), W2, b2.reshape(1, D),
                        Wa, ba.reshape(1, 1), Wx1, bx1.reshape(1, D), Wx2)
        msgp, xaccp = _scatter(m, coef.reshape(he), dbx, dby, dbz, dst_idx,
                               zm, zf)
        msgs += [msgp.reshape(NC, N, D)[0], msgp.reshape(NC, N, D)[1]]
        xaccp = xaccp.reshape(NC, N, XW)
        xas += [xaccp[0], xaccp[1]]
    xpad = jnp.pad(x, ((0, 0), (0, XW - 3)))
    h_new, xsum = _node(h, msgs, xpad, xas,
                        Wh1[:D], Wh1[D:], bh1.reshape(1, D), Wh2,
                        bh2.reshape(1, D))
    return (h_new, xsum[:, :3])
